# baseline (speedup 1.0000x reference)
import functools

import jax
import jax.numpy as jnp
from jax import lax
from jax.experimental import pallas as pl
from jax.experimental.pallas import tpu as pltpu
from jax.experimental.pallas import tpu_sc as plsc

EPS = 1e-5
NG = 64
NC = 2
NS = 16
NW = NC * NS
K = 80


def _deg_call(dst3, NP, G):
    mesh = plsc.VectorSubcoreMesh(core_axis_name="c", subcore_axis_name="s")
    rpt = NP // NS

    @functools.partial(
        pl.kernel,
        out_type=jax.ShapeDtypeStruct((NC, NP), jnp.float32),
        mesh=mesh,
        scratch_types=[
            pltpu.VMEM((G, K), jnp.int32),
            pltpu.VMEM((K,), jnp.float32),
            pltpu.VMEM((rpt,), jnp.float32),
            pltpu.VMEM_SHARED((NP,), jnp.float32),
        ],
    )
    def k(dst_hbm, out_hbm, idx_v, ones_v, buf_v, acc_sh):
        c = lax.axis_index("c")
        s = lax.axis_index("s")
        w = c * NS + s

        def fill_zero(i, carry):
            buf_v[pl.ds(i * 16, 16)] = jnp.zeros((16,), jnp.float32)
            return carry
        lax.fori_loop(0, rpt // 16, fill_zero, 0)

        def fill_one(i, carry):
            ones_v[pl.ds(i * 16, 16)] = jnp.ones((16,), jnp.float32)
            return carry
        lax.fori_loop(0, K // 16, fill_one, 0)

        pltpu.sync_copy(buf_v, acc_sh.at[pl.ds(s * rpt, rpt)])
        plsc.subcore_barrier()

        pltpu.sync_copy(dst_hbm.at[w], idx_v)

        def body(g, carry):
            pltpu.sync_copy(ones_v, acc_sh.at[idx_v.at[g]], add=True)
            return carry
        lax.fori_loop(0, G, body, 0)

        plsc.subcore_barrier()
        pltpu.sync_copy(acc_sh.at[pl.ds(s * rpt, rpt)], buf_v)
        pltpu.sync_copy(buf_v, out_hbm.at[c, pl.ds(s * rpt, rpt)])

    return k(dst3)


def _edge_aggregate_call(xws, src3, dst3, NP, G, KA, D):
    mesh = plsc.VectorSubcoreMesh(core_axis_name="c", subcore_axis_name="s")
    rpt = NP // NS
    ch = 64

    @functools.partial(
        pl.kernel,
        out_type=jax.ShapeDtypeStruct((NC, NP, D), jnp.float32),
        mesh=mesh,
        scratch_types=[
            pltpu.VMEM((G // 2, KA), jnp.int32),
            pltpu.VMEM((G // 2, KA), jnp.int32),
            pltpu.VMEM((KA, D), jnp.float32),
            pltpu.VMEM((KA, D), jnp.float32),
            pltpu.VMEM_SHARED((NP, D), jnp.float32),
            pltpu.SemaphoreType.DMA,
        ],
    )
    def k(xws_hbm, src_hbm, dst_hbm, out_hbm, src_v, dst_v, rows0_v, rows1_v, acc_sh, sem):
        c = lax.axis_index("c")
        s = lax.axis_index("s")
        w = c * NS + s
        rows = (rows0_v, rows1_v)
        G2 = G // 2

        def zrow(i, carry):
            for j in range(D // 16):
                rows0_v[i, pl.ds(j * 16, 16)] = jnp.zeros((16,), jnp.float32)
            return carry
        lax.fori_loop(0, ch, zrow, 0)

        for t in range(rpt // ch):
            pltpu.sync_copy(rows0_v.at[pl.ds(0, ch)], acc_sh.at[pl.ds(s * rpt + t * ch, ch)])
        plsc.subcore_barrier()

        for h in range(2):
            pltpu.sync_copy(src_hbm.at[w, pl.ds(h * G2, G2)], src_v)
            pltpu.sync_copy(dst_hbm.at[w, pl.ds(h * G2, G2)], dst_v)

            pltpu.async_copy(xws_hbm.at[src_v.at[0]], rows0_v, sem)

            def pair(p, carry):
                for b in range(2):
                    g = 2 * p + b

                    @pl.when(g + 1 < G2)
                    def _():
                        pltpu.async_copy(xws_hbm.at[src_v.at[g + 1]], rows[1 - b], sem)

                    pltpu.make_async_copy(xws_hbm.at[src_v.at[g]], rows[b], sem).wait()
                    pltpu.sync_copy(rows[b], acc_sh.at[dst_v.at[g]], add=True)
                return carry
            lax.fori_loop(0, G2 // 2, pair, 0)

        plsc.subcore_barrier()
        for t in range(rpt // ch):
            pltpu.sync_copy(acc_sh.at[pl.ds(s * rpt + t * ch, ch)], rows0_v.at[pl.ds(0, ch)])
            pltpu.sync_copy(rows0_v.at[pl.ds(0, ch)], out_hbm.at[c, pl.ds(s * rpt + t * ch, ch)])

    return k(xws, src3, dst3)


def _xw_scale_call(x, W, deg0, deg1, C):
    N, DIN = x.shape
    D = W.shape[1]

    def body(x_ref, w_ref, d0_ref, d1_ref, xws_ref, dis_ref):
        deg = d0_ref[...] + d1_ref[...] + 1.0
        dis = lax.rsqrt(deg)
        xw = jnp.dot(x_ref[...], w_ref[...], preferred_element_type=jnp.float32, precision=lax.Precision.HIGHEST)
        xws_ref[...] = xw * dis
        dis_ref[...] = dis

    return pl.pallas_call(
        body,
        grid=(N // C,),
        in_specs=[
            pl.BlockSpec((C, DIN), lambda i: (i, 0)),
            pl.BlockSpec((DIN, D), lambda i: (0, 0)),
            pl.BlockSpec((C, 1), lambda i: (i, 0)),
            pl.BlockSpec((C, 1), lambda i: (i, 0)),
        ],
        out_specs=[
            pl.BlockSpec((C, D), lambda i: (i, 0)),
            pl.BlockSpec((C, 1), lambda i: (i, 0)),
        ],
        out_shape=[
            jax.ShapeDtypeStruct((N, D), jnp.float32),
            jax.ShapeDtypeStruct((N, 1), jnp.float32),
        ],
    )(x, W, deg0, deg1)


def _post_agg_call(a0, a1, xws, dis, b, batch, C):
    N, D = xws.shape

    def body(a0_ref, a1_ref, xws_ref, dis_ref, b_ref, bat_ref, h_ref, S_ref, cnt_ref):
        i = pl.program_id(0)
        h = dis_ref[...] * (a0_ref[...] + a1_ref[...] + xws_ref[...]) + b_ref[...][None, :]
        h_ref[...] = h
        oh = (lax.broadcasted_iota(jnp.int32, (NG, C), 0) == bat_ref[...][:, 0][None, :]).astype(jnp.float32)

        @pl.when(i == 0)
        def _():
            S_ref[...] = jnp.zeros_like(S_ref)
            cnt_ref[...] = jnp.zeros_like(cnt_ref)

        S_ref[...] += jnp.dot(oh, h, preferred_element_type=jnp.float32, precision=lax.Precision.HIGHEST)
        cnt_ref[...] += jnp.sum(oh, axis=1)

    return pl.pallas_call(
        body,
        grid=(N // C,),
        in_specs=[
            pl.BlockSpec((C, D), lambda i: (i, 0)),
            pl.BlockSpec((C, D), lambda i: (i, 0)),
            pl.BlockSpec((C, D), lambda i: (i, 0)),
            pl.BlockSpec((C, 1), lambda i: (i, 0)),
            pl.BlockSpec((D,), lambda i: (0,)),
            pl.BlockSpec((C, 1), lambda i: (i, 0)),
        ],
        out_specs=[
            pl.BlockSpec((C, D), lambda i: (i, 0)),
            pl.BlockSpec((NG, D), lambda i: (0, 0)),
            pl.BlockSpec((NG,), lambda i: (0,)),
        ],
        out_shape=[
            jax.ShapeDtypeStruct((N, D), jnp.float32),
            jax.ShapeDtypeStruct((NG, D), jnp.float32),
            jax.ShapeDtypeStruct((NG,), jnp.float32),
        ],
    )(a0, a1, xws, dis, b, batch)


def _center_call(h, S, cnt, batch, ms, C):
    N, D = h.shape

    def body(h_ref, S_ref, cnt_ref, bat_ref, ms_ref, out_ref, V_ref):
        i = pl.program_id(0)
        mean = S_ref[...] / jnp.maximum(cnt_ref[...], 1.0)[:, None]
        bat = bat_ref[...][:, 0]
        ohT = (lax.broadcasted_iota(jnp.int32, (C, NG), 1) == bat[:, None]).astype(jnp.float32)
        mb = jnp.dot(ohT, mean, preferred_element_type=jnp.float32, precision=lax.Precision.HIGHEST)
        out = h_ref[...] - mb * ms_ref[...][None, :]
        out_ref[...] = out
        oh = (lax.broadcasted_iota(jnp.int32, (NG, C), 0) == bat[None, :]).astype(jnp.float32)

        @pl.when(i == 0)
        def _():
            V_ref[...] = jnp.zeros_like(V_ref)

        V_ref[...] += jnp.dot(oh, out * out, preferred_element_type=jnp.float32, precision=lax.Precision.HIGHEST)

    return pl.pallas_call(
        body,
        grid=(N // C,),
        in_specs=[
            pl.BlockSpec((C, D), lambda i: (i, 0)),
            pl.BlockSpec((NG, D), lambda i: (0, 0)),
            pl.BlockSpec((NG,), lambda i: (0,)),
            pl.BlockSpec((C, 1), lambda i: (i, 0)),
            pl.BlockSpec((D,), lambda i: (0,)),
        ],
        out_specs=[
            pl.BlockSpec((C, D), lambda i: (i, 0)),
            pl.BlockSpec((NG, D), lambda i: (0, 0)),
        ],
        out_shape=[
            jax.ShapeDtypeStruct((N, D), jnp.float32),
            jax.ShapeDtypeStruct((NG, D), jnp.float32),
        ],
    )(h, S, cnt, batch, ms)


def _norm_relu_xw_call(out, V, cnt, w, b, dis, W2, batch, C):
    N, D = out.shape

    def body(o_ref, V_ref, cnt_ref, w_ref, b_ref, dis_ref, W2_ref, bat_ref, xws_ref):
        var = V_ref[...] / jnp.maximum(cnt_ref[...], 1.0)[:, None]
        std = jnp.sqrt(var + EPS)
        ohT = (lax.broadcasted_iota(jnp.int32, (C, NG), 1) == bat_ref[...][:, 0][:, None]).astype(jnp.float32)
        stdb = jnp.dot(ohT, std, preferred_element_type=jnp.float32, precision=lax.Precision.HIGHEST)
        hn = w_ref[...][None, :] * o_ref[...] / stdb + b_ref[...][None, :]
        hn = jnp.maximum(hn, 0.0)
        xw = jnp.dot(hn, W2_ref[...], preferred_element_type=jnp.float32, precision=lax.Precision.HIGHEST)
        xws_ref[...] = xw * dis_ref[...]

    return pl.pallas_call(
        body,
        grid=(N // C,),
        in_specs=[
            pl.BlockSpec((C, D), lambda i: (i, 0)),
            pl.BlockSpec((NG, D), lambda i: (0, 0)),
            pl.BlockSpec((NG,), lambda i: (0,)),
            pl.BlockSpec((D,), lambda i: (0,)),
            pl.BlockSpec((D,), lambda i: (0,)),
            pl.BlockSpec((C, 1), lambda i: (i, 0)),
            pl.BlockSpec((D, D), lambda i: (0, 0)),
            pl.BlockSpec((C, 1), lambda i: (i, 0)),
        ],
        out_specs=pl.BlockSpec((C, D), lambda i: (i, 0)),
        out_shape=jax.ShapeDtypeStruct((N, D), jnp.float32),
    )(out, V, cnt, w, b, dis, W2, batch)


def _norm_relu_pool_call(out, V, cnt, w, b, batch, C):
    N, D = out.shape

    def body(o_ref, V_ref, cnt_ref, w_ref, b_ref, bat_ref, P_ref):
        i = pl.program_id(0)
        var = V_ref[...] / jnp.maximum(cnt_ref[...], 1.0)[:, None]
        std = jnp.sqrt(var + EPS)
        bat = bat_ref[...][:, 0]
        ohT = (lax.broadcasted_iota(jnp.int32, (C, NG), 1) == bat[:, None]).astype(jnp.float32)
        stdb = jnp.dot(ohT, std, preferred_element_type=jnp.float32, precision=lax.Precision.HIGHEST)
        hn = w_ref[...][None, :] * o_ref[...] / stdb + b_ref[...][None, :]
        hn = jnp.maximum(hn, 0.0)
        oh = (lax.broadcasted_iota(jnp.int32, (NG, C), 0) == bat[None, :]).astype(jnp.float32)

        @pl.when(i == 0)
        def _():
            P_ref[...] = jnp.zeros_like(P_ref)

        P_ref[...] += jnp.dot(oh, hn, preferred_element_type=jnp.float32, precision=lax.Precision.HIGHEST)

    return pl.pallas_call(
        body,
        grid=(N // C,),
        in_specs=[
            pl.BlockSpec((C, D), lambda i: (i, 0)),
            pl.BlockSpec((NG, D), lambda i: (0, 0)),
            pl.BlockSpec((NG,), lambda i: (0,)),
            pl.BlockSpec((D,), lambda i: (0,)),
            pl.BlockSpec((D,), lambda i: (0,)),
            pl.BlockSpec((C, 1), lambda i: (i, 0)),
        ],
        out_specs=pl.BlockSpec((NG, D), lambda i: (0, 0)),
        out_shape=jax.ShapeDtypeStruct((NG, D), jnp.float32),
    )(out, V, cnt, w, b, batch)


def _final_call(POOL, cnt, lin_W, lin_b):
    D = POOL.shape[1]
    NCLS = lin_W.shape[1]

    def body(P_ref, cnt_ref, W_ref, b_ref, o_ref):
        pooled = P_ref[...] / jnp.maximum(cnt_ref[...], 1.0)[:, None]
        o_ref[...] = jnp.dot(pooled, W_ref[...], preferred_element_type=jnp.float32, precision=lax.Precision.HIGHEST) + b_ref[...][None, :]

    return pl.pallas_call(
        body,
        in_specs=[
            pl.BlockSpec((NG, D), lambda: (0, 0)),
            pl.BlockSpec((NG,), lambda: (0,)),
            pl.BlockSpec((D, NCLS), lambda: (0, 0)),
            pl.BlockSpec((NCLS,), lambda: (0,)),
        ],
        out_specs=pl.BlockSpec((NG, NCLS), lambda: (0, 0)),
        out_shape=jax.ShapeDtypeStruct((NG, NCLS), jnp.float32),
    )(POOL, cnt, lin_W, lin_b)


def kernel(x, edge_index, batch, W1, b1, gn1_weight, gn1_bias, gn1_mean_scale,
           W2, b2, gn2_weight, gn2_bias, gn2_mean_scale, lin_W, lin_b):
    N, DIN = x.shape
    D = W1.shape[1]
    E = edge_index.shape[1]
    C = 1000

    Gd = E // (NW * K)
    NPd = ((N + (16 * NS) - 1) // (16 * NS)) * (16 * NS)
    dst3d = edge_index[1].reshape(NW, Gd, K)

    KA = 64
    Ga = -(-E // (NW * KA))
    Ga = -(-Ga // 4) * 4
    Ea = NW * KA * Ga
    pad = Ea - E
    NPa = -(-N // (NS * 64)) * (NS * 64)
    src_p = jnp.concatenate([edge_index[0], jnp.zeros((pad,), jnp.int32)])
    dst_p = jnp.concatenate(
        [edge_index[1], N + (jnp.arange(pad, dtype=jnp.int32) % (NPa - N))])
    src3a = src_p.reshape(NW, Ga, KA)
    dst3a = dst_p.reshape(NW, Ga, KA)

    degp = _deg_call(dst3d, NPd, Gd)
    deg0, deg1 = degp[0, :N, None], degp[1, :N, None]

    batch2 = batch[:, None]
    xws1, dis = _xw_scale_call(x, W1, deg0, deg1, C)

    aggp1 = _edge_aggregate_call(xws1, src3a, dst3a, NPa, Ga, KA, D)
    h1, S1, cnt = _post_agg_call(aggp1[0], aggp1[1], xws1, dis, b1, batch2, C)
    out1, V1 = _center_call(h1, S1, cnt, batch2, gn1_mean_scale, C)
    xws2 = _norm_relu_xw_call(out1, V1, cnt, gn1_weight, gn1_bias, dis, W2, batch2, C)

    aggp2 = _edge_aggregate_call(xws2, src3a, dst3a, NPa, Ga, KA, D)
    h2, S2, cnt2 = _post_agg_call(aggp2[0], aggp2[1], xws2, dis, b2, batch2, C)
    out2, V2 = _center_call(h2, S2, cnt2, batch2, gn2_mean_scale, C)
    POOL = _norm_relu_pool_call(out2, V2, cnt2, gn2_weight, gn2_bias, batch2, C)

    return _final_call(POOL, cnt2, lin_W, lin_b)


# speedup vs baseline: 1.0636x; 1.0636x over previous
import functools

import jax
import jax.numpy as jnp
from jax import lax
from jax.experimental import pallas as pl
from jax.experimental.pallas import tpu as pltpu
from jax.experimental.pallas import tpu_sc as plsc

EPS = 1e-5
NG = 64
NC = 2
NS = 16
NW = NC * NS
K = 80


def _deg_call(dst3, NP, G):
    mesh = plsc.VectorSubcoreMesh(core_axis_name="c", subcore_axis_name="s")
    rpt = NP // NS

    @functools.partial(
        pl.kernel,
        out_type=jax.ShapeDtypeStruct((NC, NP), jnp.float32),
        mesh=mesh,
        scratch_types=[
            pltpu.VMEM((G, K), jnp.int32),
            pltpu.VMEM((K,), jnp.float32),
            pltpu.VMEM((rpt,), jnp.float32),
            pltpu.VMEM_SHARED((NP,), jnp.float32),
        ],
    )
    def k(dst_hbm, out_hbm, idx_v, ones_v, buf_v, acc_sh):
        c = lax.axis_index("c")
        s = lax.axis_index("s")
        w = c * NS + s

        def fill_zero(i, carry):
            buf_v[pl.ds(i * 16, 16)] = jnp.zeros((16,), jnp.float32)
            return carry
        lax.fori_loop(0, rpt // 16, fill_zero, 0)

        def fill_one(i, carry):
            ones_v[pl.ds(i * 16, 16)] = jnp.ones((16,), jnp.float32)
            return carry
        lax.fori_loop(0, K // 16, fill_one, 0)

        pltpu.sync_copy(buf_v, acc_sh.at[pl.ds(s * rpt, rpt)])
        plsc.subcore_barrier()

        pltpu.sync_copy(dst_hbm.at[w], idx_v)

        def body(g, carry):
            pltpu.sync_copy(ones_v, acc_sh.at[idx_v.at[g]], add=True)
            return carry
        lax.fori_loop(0, G, body, 0)

        plsc.subcore_barrier()
        pltpu.sync_copy(acc_sh.at[pl.ds(s * rpt, rpt)], buf_v)
        pltpu.sync_copy(buf_v, out_hbm.at[c, pl.ds(s * rpt, rpt)])

    return k(dst3)


def _edge_aggregate_call(xws, src3, dst3, NP, G, KA, D):
    mesh = plsc.VectorSubcoreMesh(core_axis_name="c", subcore_axis_name="s")
    rpt = NP // NS
    ch = 64

    @functools.partial(
        pl.kernel,
        out_type=jax.ShapeDtypeStruct((NC, NP, D), jnp.float32),
        mesh=mesh,
        scratch_types=[
            pltpu.VMEM((G // 2, KA), jnp.int32),
            pltpu.VMEM((G // 2, KA), jnp.int32),
            pltpu.VMEM((KA, D), jnp.float32),
            pltpu.VMEM((KA, D), jnp.float32),
            pltpu.VMEM_SHARED((NP, D), jnp.float32),
            pltpu.SemaphoreType.DMA,
        ],
    )
    def k(xws_hbm, src_hbm, dst_hbm, out_hbm, src_v, dst_v, rows0_v, rows1_v, acc_sh, sem):
        c = lax.axis_index("c")
        s = lax.axis_index("s")
        w = c * NS + s
        rows = (rows0_v, rows1_v)
        G2 = G // 2

        def zrow(i, carry):
            for j in range(D // 16):
                rows0_v[i, pl.ds(j * 16, 16)] = jnp.zeros((16,), jnp.float32)
            return carry
        lax.fori_loop(0, ch, zrow, 0)

        for t in range(rpt // ch):
            pltpu.sync_copy(rows0_v.at[pl.ds(0, ch)], acc_sh.at[pl.ds(s * rpt + t * ch, ch)])
        plsc.subcore_barrier()

        for h in range(2):
            pltpu.sync_copy(src_hbm.at[w, pl.ds(h * G2, G2)], src_v)
            pltpu.sync_copy(dst_hbm.at[w, pl.ds(h * G2, G2)], dst_v)

            pltpu.async_copy(xws_hbm.at[src_v.at[0]], rows0_v, sem)

            def pair(p, carry):
                for b in range(2):
                    g = 2 * p + b

                    @pl.when(g + 1 < G2)
                    def _():
                        pltpu.async_copy(xws_hbm.at[src_v.at[g + 1]], rows[1 - b], sem)

                    pltpu.make_async_copy(xws_hbm.at[src_v.at[g]], rows[b], sem).wait()
                    pltpu.sync_copy(rows[b], acc_sh.at[dst_v.at[g]], add=True)
                return carry
            lax.fori_loop(0, G2 // 2, pair, 0)

        plsc.subcore_barrier()
        for t in range(rpt // ch):
            pltpu.sync_copy(acc_sh.at[pl.ds(s * rpt + t * ch, ch)], rows0_v.at[pl.ds(0, ch)])
            pltpu.sync_copy(rows0_v.at[pl.ds(0, ch)], out_hbm.at[c, pl.ds(s * rpt + t * ch, ch)])

    return k(xws, src3, dst3)


def _xw_scale_call(x, W, deg0, deg1, C):
    N, DIN = x.shape
    D = W.shape[1]

    def body(x_ref, w_ref, d0_ref, d1_ref, xws_ref, dis_ref):
        deg = d0_ref[...] + d1_ref[...] + 1.0
        dis = lax.rsqrt(deg)
        xw = jnp.dot(x_ref[...], w_ref[...], preferred_element_type=jnp.float32, precision=lax.Precision.HIGHEST)
        xws_ref[...] = xw * dis
        dis_ref[...] = dis

    return pl.pallas_call(
        body,
        grid=(N // C,),
        in_specs=[
            pl.BlockSpec((C, DIN), lambda i: (i, 0)),
            pl.BlockSpec((DIN, D), lambda i: (0, 0)),
            pl.BlockSpec((C, 1), lambda i: (i, 0)),
            pl.BlockSpec((C, 1), lambda i: (i, 0)),
        ],
        out_specs=[
            pl.BlockSpec((C, D), lambda i: (i, 0)),
            pl.BlockSpec((C, 1), lambda i: (i, 0)),
        ],
        out_shape=[
            jax.ShapeDtypeStruct((N, D), jnp.float32),
            jax.ShapeDtypeStruct((N, 1), jnp.float32),
        ],
    )(x, W, deg0, deg1)


def _post_agg_call(a0, a1, xws, dis, b, batch, C):
    N, D = xws.shape

    def body(a0_ref, a1_ref, xws_ref, dis_ref, b_ref, bat_ref, h_ref, S_ref, cnt_ref):
        i = pl.program_id(0)
        h = dis_ref[...] * (a0_ref[...] + a1_ref[...] + xws_ref[...]) + b_ref[...][None, :]
        h_ref[...] = h
        oh = (lax.broadcasted_iota(jnp.int32, (NG, C), 0) == bat_ref[...][:, 0][None, :]).astype(jnp.float32)

        @pl.when(i == 0)
        def _():
            S_ref[...] = jnp.zeros_like(S_ref)
            cnt_ref[...] = jnp.zeros_like(cnt_ref)

        S_ref[...] += jnp.dot(oh, h, preferred_element_type=jnp.float32, precision=lax.Precision.HIGHEST)
        cnt_ref[...] += jnp.sum(oh, axis=1)

    return pl.pallas_call(
        body,
        grid=(N // C,),
        in_specs=[
            pl.BlockSpec((C, D), lambda i: (i, 0)),
            pl.BlockSpec((C, D), lambda i: (i, 0)),
            pl.BlockSpec((C, D), lambda i: (i, 0)),
            pl.BlockSpec((C, 1), lambda i: (i, 0)),
            pl.BlockSpec((D,), lambda i: (0,)),
            pl.BlockSpec((C, 1), lambda i: (i, 0)),
        ],
        out_specs=[
            pl.BlockSpec((C, D), lambda i: (i, 0)),
            pl.BlockSpec((NG, D), lambda i: (0, 0)),
            pl.BlockSpec((NG,), lambda i: (0,)),
        ],
        out_shape=[
            jax.ShapeDtypeStruct((N, D), jnp.float32),
            jax.ShapeDtypeStruct((NG, D), jnp.float32),
            jax.ShapeDtypeStruct((NG,), jnp.float32),
        ],
    )(a0, a1, xws, dis, b, batch)


def _center_call(h, S, cnt, batch, ms, C):
    N, D = h.shape

    def body(h_ref, S_ref, cnt_ref, bat_ref, ms_ref, out_ref, V_ref):
        i = pl.program_id(0)
        mean = S_ref[...] / jnp.maximum(cnt_ref[...], 1.0)[:, None]
        bat = bat_ref[...][:, 0]
        ohT = (lax.broadcasted_iota(jnp.int32, (C, NG), 1) == bat[:, None]).astype(jnp.float32)
        mb = jnp.dot(ohT, mean, preferred_element_type=jnp.float32, precision=lax.Precision.HIGHEST)
        out = h_ref[...] - mb * ms_ref[...][None, :]
        out_ref[...] = out
        oh = (lax.broadcasted_iota(jnp.int32, (NG, C), 0) == bat[None, :]).astype(jnp.float32)

        @pl.when(i == 0)
        def _():
            V_ref[...] = jnp.zeros_like(V_ref)

        V_ref[...] += jnp.dot(oh, out * out, preferred_element_type=jnp.float32, precision=lax.Precision.HIGHEST)

    return pl.pallas_call(
        body,
        grid=(N // C,),
        in_specs=[
            pl.BlockSpec((C, D), lambda i: (i, 0)),
            pl.BlockSpec((NG, D), lambda i: (0, 0)),
            pl.BlockSpec((NG,), lambda i: (0,)),
            pl.BlockSpec((C, 1), lambda i: (i, 0)),
            pl.BlockSpec((D,), lambda i: (0,)),
        ],
        out_specs=[
            pl.BlockSpec((C, D), lambda i: (i, 0)),
            pl.BlockSpec((NG, D), lambda i: (0, 0)),
        ],
        out_shape=[
            jax.ShapeDtypeStruct((N, D), jnp.float32),
            jax.ShapeDtypeStruct((NG, D), jnp.float32),
        ],
    )(h, S, cnt, batch, ms)


def _norm_relu_xw_call(out, V, cnt, w, b, dis, W2, batch, C):
    N, D = out.shape

    def body(o_ref, V_ref, cnt_ref, w_ref, b_ref, dis_ref, W2_ref, bat_ref, xws_ref):
        var = V_ref[...] / jnp.maximum(cnt_ref[...], 1.0)[:, None]
        std = jnp.sqrt(var + EPS)
        ohT = (lax.broadcasted_iota(jnp.int32, (C, NG), 1) == bat_ref[...][:, 0][:, None]).astype(jnp.float32)
        stdb = jnp.dot(ohT, std, preferred_element_type=jnp.float32, precision=lax.Precision.HIGHEST)
        hn = w_ref[...][None, :] * o_ref[...] / stdb + b_ref[...][None, :]
        hn = jnp.maximum(hn, 0.0)
        xw = jnp.dot(hn, W2_ref[...], preferred_element_type=jnp.float32, precision=lax.Precision.HIGHEST)
        xws_ref[...] = xw * dis_ref[...]

    return pl.pallas_call(
        body,
        grid=(N // C,),
        in_specs=[
            pl.BlockSpec((C, D), lambda i: (i, 0)),
            pl.BlockSpec((NG, D), lambda i: (0, 0)),
            pl.BlockSpec((NG,), lambda i: (0,)),
            pl.BlockSpec((D,), lambda i: (0,)),
            pl.BlockSpec((D,), lambda i: (0,)),
            pl.BlockSpec((C, 1), lambda i: (i, 0)),
            pl.BlockSpec((D, D), lambda i: (0, 0)),
            pl.BlockSpec((C, 1), lambda i: (i, 0)),
        ],
        out_specs=pl.BlockSpec((C, D), lambda i: (i, 0)),
        out_shape=jax.ShapeDtypeStruct((N, D), jnp.float32),
    )(out, V, cnt, w, b, dis, W2, batch)


def _norm_relu_pool_call(out, V, cnt, w, b, batch, C):
    N, D = out.shape

    def body(o_ref, V_ref, cnt_ref, w_ref, b_ref, bat_ref, P_ref):
        i = pl.program_id(0)
        var = V_ref[...] / jnp.maximum(cnt_ref[...], 1.0)[:, None]
        std = jnp.sqrt(var + EPS)
        bat = bat_ref[...][:, 0]
        ohT = (lax.broadcasted_iota(jnp.int32, (C, NG), 1) == bat[:, None]).astype(jnp.float32)
        stdb = jnp.dot(ohT, std, preferred_element_type=jnp.float32, precision=lax.Precision.HIGHEST)
        hn = w_ref[...][None, :] * o_ref[...] / stdb + b_ref[...][None, :]
        hn = jnp.maximum(hn, 0.0)
        oh = (lax.broadcasted_iota(jnp.int32, (NG, C), 0) == bat[None, :]).astype(jnp.float32)

        @pl.when(i == 0)
        def _():
            P_ref[...] = jnp.zeros_like(P_ref)

        P_ref[...] += jnp.dot(oh, hn, preferred_element_type=jnp.float32, precision=lax.Precision.HIGHEST)

    return pl.pallas_call(
        body,
        grid=(N // C,),
        in_specs=[
            pl.BlockSpec((C, D), lambda i: (i, 0)),
            pl.BlockSpec((NG, D), lambda i: (0, 0)),
            pl.BlockSpec((NG,), lambda i: (0,)),
            pl.BlockSpec((D,), lambda i: (0,)),
            pl.BlockSpec((D,), lambda i: (0,)),
            pl.BlockSpec((C, 1), lambda i: (i, 0)),
        ],
        out_specs=pl.BlockSpec((NG, D), lambda i: (0, 0)),
        out_shape=jax.ShapeDtypeStruct((NG, D), jnp.float32),
    )(out, V, cnt, w, b, batch)


def _final_call(POOL, cnt, lin_W, lin_b):
    D = POOL.shape[1]
    NCLS = lin_W.shape[1]

    def body(P_ref, cnt_ref, W_ref, b_ref, o_ref):
        pooled = P_ref[...] / jnp.maximum(cnt_ref[...], 1.0)[:, None]
        o_ref[...] = jnp.dot(pooled, W_ref[...], preferred_element_type=jnp.float32, precision=lax.Precision.HIGHEST) + b_ref[...][None, :]

    return pl.pallas_call(
        body,
        in_specs=[
            pl.BlockSpec((NG, D), lambda: (0, 0)),
            pl.BlockSpec((NG,), lambda: (0,)),
            pl.BlockSpec((D, NCLS), lambda: (0, 0)),
            pl.BlockSpec((NCLS,), lambda: (0,)),
        ],
        out_specs=pl.BlockSpec((NG, NCLS), lambda: (0, 0)),
        out_shape=jax.ShapeDtypeStruct((NG, NCLS), jnp.float32),
    )(POOL, cnt, lin_W, lin_b)


def kernel(x, edge_index, batch, W1, b1, gn1_weight, gn1_bias, gn1_mean_scale,
           W2, b2, gn2_weight, gn2_bias, gn2_mean_scale, lin_W, lin_b):
    N, DIN = x.shape
    D = W1.shape[1]
    E = edge_index.shape[1]
    C = 1000

    Gd = E // (NW * K)
    NPd = ((N + (16 * NS) - 1) // (16 * NS)) * (16 * NS)
    dst3d = edge_index[1].reshape(NW, Gd, K)

    KA = 128
    Ga = -(-E // (NW * KA))
    Ga = -(-Ga // 4) * 4
    Ea = NW * KA * Ga
    pad = Ea - E
    NPa = -(-N // (NS * 64)) * (NS * 64)
    src_p = jnp.concatenate([edge_index[0], jnp.zeros((pad,), jnp.int32)])
    dst_p = jnp.concatenate(
        [edge_index[1], N + (jnp.arange(pad, dtype=jnp.int32) % (NPa - N))])
    src3a = src_p.reshape(NW, Ga, KA)
    dst3a = dst_p.reshape(NW, Ga, KA)

    degp = _deg_call(dst3d, NPd, Gd)
    deg0, deg1 = degp[0, :N, None], degp[1, :N, None]

    batch2 = batch[:, None]
    xws1, dis = _xw_scale_call(x, W1, deg0, deg1, C)

    aggp1 = _edge_aggregate_call(xws1, src3a, dst3a, NPa, Ga, KA, D)
    h1, S1, cnt = _post_agg_call(aggp1[0], aggp1[1], xws1, dis, b1, batch2, C)
    out1, V1 = _center_call(h1, S1, cnt, batch2, gn1_mean_scale, C)
    xws2 = _norm_relu_xw_call(out1, V1, cnt, gn1_weight, gn1_bias, dis, W2, batch2, C)

    aggp2 = _edge_aggregate_call(xws2, src3a, dst3a, NPa, Ga, KA, D)
    h2, S2, cnt2 = _post_agg_call(aggp2[0], aggp2[1], xws2, dis, b2, batch2, C)
    out2, V2 = _center_call(h2, S2, cnt2, batch2, gn2_mean_scale, C)
    POOL = _norm_relu_pool_call(out2, V2, cnt2, gn2_weight, gn2_bias, batch2, C)

    return _final_call(POOL, cnt2, lin_W, lin_b)


# speedup vs baseline: 2.5538x; 2.4011x over previous
import functools

import jax
import jax.numpy as jnp
from jax import lax
from jax.experimental import pallas as pl
from jax.experimental.pallas import tpu as pltpu
from jax.experimental.pallas import tpu_sc as plsc

EPS = 1e-5
NG = 64
NC = 2
NS = 16
NW = NC * NS
K = 80


def _deg_call(dst3, NP, G):
    mesh = plsc.VectorSubcoreMesh(core_axis_name="c", subcore_axis_name="s")
    rpt = NP // NS

    @functools.partial(
        pl.kernel,
        out_type=jax.ShapeDtypeStruct((NC, NP), jnp.float32),
        mesh=mesh,
        scratch_types=[
            pltpu.VMEM((G, K), jnp.int32),
            pltpu.VMEM((K,), jnp.float32),
            pltpu.VMEM((rpt,), jnp.float32),
            pltpu.VMEM_SHARED((NP,), jnp.float32),
        ],
    )
    def k(dst_hbm, out_hbm, idx_v, ones_v, buf_v, acc_sh):
        c = lax.axis_index("c")
        s = lax.axis_index("s")
        w = c * NS + s

        def fill_zero(i, carry):
            buf_v[pl.ds(i * 16, 16)] = jnp.zeros((16,), jnp.float32)
            return carry
        lax.fori_loop(0, rpt // 16, fill_zero, 0)

        def fill_one(i, carry):
            ones_v[pl.ds(i * 16, 16)] = jnp.ones((16,), jnp.float32)
            return carry
        lax.fori_loop(0, K // 16, fill_one, 0)

        pltpu.sync_copy(buf_v, acc_sh.at[pl.ds(s * rpt, rpt)])
        plsc.subcore_barrier()

        pltpu.sync_copy(dst_hbm.at[w], idx_v)

        def body(g, carry):
            pltpu.sync_copy(ones_v, acc_sh.at[idx_v.at[g]], add=True)
            return carry
        lax.fori_loop(0, G, body, 0)

        plsc.subcore_barrier()
        pltpu.sync_copy(acc_sh.at[pl.ds(s * rpt, rpt)], buf_v)
        pltpu.sync_copy(buf_v, out_hbm.at[c, pl.ds(s * rpt, rpt)])

    return k(dst3)


def _edge_aggregate_call(xws, src3, dst3, NP, G, KA, D):
    mesh = plsc.VectorSubcoreMesh(core_axis_name="c", subcore_axis_name="s")
    rpt = NP // NS
    ch = 64

    @functools.partial(
        pl.kernel,
        out_type=jax.ShapeDtypeStruct((NC, NP, D), jnp.float32),
        mesh=mesh,
        scratch_types=[
            pltpu.VMEM((G // 2, KA), jnp.int32),
            pltpu.VMEM((G // 2, KA), jnp.int32),
            pltpu.VMEM((KA, D), jnp.float32),
            pltpu.VMEM((KA, D), jnp.float32),
            pltpu.VMEM_SHARED((NP, D), jnp.float32),
            pltpu.SemaphoreType.DMA,
            pltpu.SemaphoreType.DMA,
        ],
    )
    def k(xws_hbm, src_hbm, dst_hbm, out_hbm, src_v, dst_v, r0, r1,
          acc_sh, gsem, ssem):
        c = lax.axis_index("c")
        s = lax.axis_index("s")
        w = c * NS + s
        rows = (r0, r1)
        G2 = G // 2

        def zrow(i, carry):
            for j in range(D // 16):
                r0[i, pl.ds(j * 16, 16)] = jnp.zeros((16,), jnp.float32)
            return carry
        lax.fori_loop(0, ch, zrow, 0)

        for t in range(rpt // ch):
            pltpu.sync_copy(r0.at[pl.ds(0, ch)], acc_sh.at[pl.ds(s * rpt + t * ch, ch)])
        plsc.subcore_barrier()

        for h in range(2):
            pltpu.sync_copy(src_hbm.at[w, pl.ds(h * G2, G2)], src_v)
            pltpu.sync_copy(dst_hbm.at[w, pl.ds(h * G2, G2)], dst_v)

            pltpu.async_copy(xws_hbm.at[src_v.at[0]], rows[0], gsem)

            def pair(p, carry):
                for b in range(2):
                    g = 2 * p + b
                    pltpu.make_async_copy(xws_hbm.at[src_v.at[g]], rows[b], gsem).wait()
                    pltpu.async_copy(rows[b], acc_sh.at[dst_v.at[g]], ssem, add=True)

                    @pl.when(g + 1 < G2)
                    def _():
                        @pl.when(g >= 1)
                        def _():
                            pltpu.make_async_copy(
                                rows[1 - b], acc_sh.at[dst_v.at[g - 1]], ssem
                            ).wait()
                        pltpu.async_copy(xws_hbm.at[src_v.at[g + 1]], rows[1 - b], gsem)
                return carry
            lax.fori_loop(0, G2 // 2, pair, 0)

            pltpu.make_async_copy(rows[G2 % 2], acc_sh.at[dst_v.at[G2 - 2]], ssem).wait()
            pltpu.make_async_copy(rows[1 - G2 % 2], acc_sh.at[dst_v.at[G2 - 1]], ssem).wait()

        plsc.subcore_barrier()
        for t in range(rpt // ch):
            pltpu.sync_copy(acc_sh.at[pl.ds(s * rpt + t * ch, ch)], r0.at[pl.ds(0, ch)])
            pltpu.sync_copy(r0.at[pl.ds(0, ch)], out_hbm.at[c, pl.ds(s * rpt + t * ch, ch)])

    return k(xws, src3, dst3)


def _xw_scale_call(x, W, deg0, deg1, C):
    N, DIN = x.shape
    D = W.shape[1]

    def body(x_ref, w_ref, d0_ref, d1_ref, xws_ref, dis_ref):
        deg = d0_ref[...] + d1_ref[...] + 1.0
        dis = lax.rsqrt(deg)
        xw = jnp.dot(x_ref[...], w_ref[...], preferred_element_type=jnp.float32, precision=lax.Precision.HIGHEST)
        xws_ref[...] = xw * dis
        dis_ref[...] = dis

    return pl.pallas_call(
        body,
        grid=(N // C,),
        in_specs=[
            pl.BlockSpec((C, DIN), lambda i: (i, 0)),
            pl.BlockSpec((DIN, D), lambda i: (0, 0)),
            pl.BlockSpec((C, 1), lambda i: (i, 0)),
            pl.BlockSpec((C, 1), lambda i: (i, 0)),
        ],
        out_specs=[
            pl.BlockSpec((C, D), lambda i: (i, 0)),
            pl.BlockSpec((C, 1), lambda i: (i, 0)),
        ],
        out_shape=[
            jax.ShapeDtypeStruct((N, D), jnp.float32),
            jax.ShapeDtypeStruct((N, 1), jnp.float32),
        ],
    )(x, W, deg0, deg1)


def _post_agg_call(a0, a1, xws, dis, b, batch, C):
    N, D = xws.shape

    def body(a0_ref, a1_ref, xws_ref, dis_ref, b_ref, bat_ref, h_ref, S_ref, cnt_ref):
        i = pl.program_id(0)
        h = dis_ref[...] * (a0_ref[...] + a1_ref[...] + xws_ref[...]) + b_ref[...][None, :]
        h_ref[...] = h
        oh = (lax.broadcasted_iota(jnp.int32, (NG, C), 0) == bat_ref[...][:, 0][None, :]).astype(jnp.float32)

        @pl.when(i == 0)
        def _():
            S_ref[...] = jnp.zeros_like(S_ref)
            cnt_ref[...] = jnp.zeros_like(cnt_ref)

        S_ref[...] += jnp.dot(oh, h, preferred_element_type=jnp.float32, precision=lax.Precision.HIGHEST)
        cnt_ref[...] += jnp.sum(oh, axis=1)

    return pl.pallas_call(
        body,
        grid=(N // C,),
        in_specs=[
            pl.BlockSpec((C, D), lambda i: (i, 0)),
            pl.BlockSpec((C, D), lambda i: (i, 0)),
            pl.BlockSpec((C, D), lambda i: (i, 0)),
            pl.BlockSpec((C, 1), lambda i: (i, 0)),
            pl.BlockSpec((D,), lambda i: (0,)),
            pl.BlockSpec((C, 1), lambda i: (i, 0)),
        ],
        out_specs=[
            pl.BlockSpec((C, D), lambda i: (i, 0)),
            pl.BlockSpec((NG, D), lambda i: (0, 0)),
            pl.BlockSpec((NG,), lambda i: (0,)),
        ],
        out_shape=[
            jax.ShapeDtypeStruct((N, D), jnp.float32),
            jax.ShapeDtypeStruct((NG, D), jnp.float32),
            jax.ShapeDtypeStruct((NG,), jnp.float32),
        ],
    )(a0, a1, xws, dis, b, batch)


def _center_call(h, S, cnt, batch, ms, C):
    N, D = h.shape

    def body(h_ref, S_ref, cnt_ref, bat_ref, ms_ref, out_ref, V_ref):
        i = pl.program_id(0)
        mean = S_ref[...] / jnp.maximum(cnt_ref[...], 1.0)[:, None]
        bat = bat_ref[...][:, 0]
        ohT = (lax.broadcasted_iota(jnp.int32, (C, NG), 1) == bat[:, None]).astype(jnp.float32)
        mb = jnp.dot(ohT, mean, preferred_element_type=jnp.float32, precision=lax.Precision.HIGHEST)
        out = h_ref[...] - mb * ms_ref[...][None, :]
        out_ref[...] = out
        oh = (lax.broadcasted_iota(jnp.int32, (NG, C), 0) == bat[None, :]).astype(jnp.float32)

        @pl.when(i == 0)
        def _():
            V_ref[...] = jnp.zeros_like(V_ref)

        V_ref[...] += jnp.dot(oh, out * out, preferred_element_type=jnp.float32, precision=lax.Precision.HIGHEST)

    return pl.pallas_call(
        body,
        grid=(N // C,),
        in_specs=[
            pl.BlockSpec((C, D), lambda i: (i, 0)),
            pl.BlockSpec((NG, D), lambda i: (0, 0)),
            pl.BlockSpec((NG,), lambda i: (0,)),
            pl.BlockSpec((C, 1), lambda i: (i, 0)),
            pl.BlockSpec((D,), lambda i: (0,)),
        ],
        out_specs=[
            pl.BlockSpec((C, D), lambda i: (i, 0)),
            pl.BlockSpec((NG, D), lambda i: (0, 0)),
        ],
        out_shape=[
            jax.ShapeDtypeStruct((N, D), jnp.float32),
            jax.ShapeDtypeStruct((NG, D), jnp.float32),
        ],
    )(h, S, cnt, batch, ms)


def _norm_relu_xw_call(out, V, cnt, w, b, dis, W2, batch, C):
    N, D = out.shape

    def body(o_ref, V_ref, cnt_ref, w_ref, b_ref, dis_ref, W2_ref, bat_ref, xws_ref):
        var = V_ref[...] / jnp.maximum(cnt_ref[...], 1.0)[:, None]
        std = jnp.sqrt(var + EPS)
        ohT = (lax.broadcasted_iota(jnp.int32, (C, NG), 1) == bat_ref[...][:, 0][:, None]).astype(jnp.float32)
        stdb = jnp.dot(ohT, std, preferred_element_type=jnp.float32, precision=lax.Precision.HIGHEST)
        hn = w_ref[...][None, :] * o_ref[...] / stdb + b_ref[...][None, :]
        hn = jnp.maximum(hn, 0.0)
        xw = jnp.dot(hn, W2_ref[...], preferred_element_type=jnp.float32, precision=lax.Precision.HIGHEST)
        xws_ref[...] = xw * dis_ref[...]

    return pl.pallas_call(
        body,
        grid=(N // C,),
        in_specs=[
            pl.BlockSpec((C, D), lambda i: (i, 0)),
            pl.BlockSpec((NG, D), lambda i: (0, 0)),
            pl.BlockSpec((NG,), lambda i: (0,)),
            pl.BlockSpec((D,), lambda i: (0,)),
            pl.BlockSpec((D,), lambda i: (0,)),
            pl.BlockSpec((C, 1), lambda i: (i, 0)),
            pl.BlockSpec((D, D), lambda i: (0, 0)),
            pl.BlockSpec((C, 1), lambda i: (i, 0)),
        ],
        out_specs=pl.BlockSpec((C, D), lambda i: (i, 0)),
        out_shape=jax.ShapeDtypeStruct((N, D), jnp.float32),
    )(out, V, cnt, w, b, dis, W2, batch)


def _norm_relu_pool_call(out, V, cnt, w, b, batch, C):
    N, D = out.shape

    def body(o_ref, V_ref, cnt_ref, w_ref, b_ref, bat_ref, P_ref):
        i = pl.program_id(0)
        var = V_ref[...] / jnp.maximum(cnt_ref[...], 1.0)[:, None]
        std = jnp.sqrt(var + EPS)
        bat = bat_ref[...][:, 0]
        ohT = (lax.broadcasted_iota(jnp.int32, (C, NG), 1) == bat[:, None]).astype(jnp.float32)
        stdb = jnp.dot(ohT, std, preferred_element_type=jnp.float32, precision=lax.Precision.HIGHEST)
        hn = w_ref[...][None, :] * o_ref[...] / stdb + b_ref[...][None, :]
        hn = jnp.maximum(hn, 0.0)
        oh = (lax.broadcasted_iota(jnp.int32, (NG, C), 0) == bat[None, :]).astype(jnp.float32)

        @pl.when(i == 0)
        def _():
            P_ref[...] = jnp.zeros_like(P_ref)

        P_ref[...] += jnp.dot(oh, hn, preferred_element_type=jnp.float32, precision=lax.Precision.HIGHEST)

    return pl.pallas_call(
        body,
        grid=(N // C,),
        in_specs=[
            pl.BlockSpec((C, D), lambda i: (i, 0)),
            pl.BlockSpec((NG, D), lambda i: (0, 0)),
            pl.BlockSpec((NG,), lambda i: (0,)),
            pl.BlockSpec((D,), lambda i: (0,)),
            pl.BlockSpec((D,), lambda i: (0,)),
            pl.BlockSpec((C, 1), lambda i: (i, 0)),
        ],
        out_specs=pl.BlockSpec((NG, D), lambda i: (0, 0)),
        out_shape=jax.ShapeDtypeStruct((NG, D), jnp.float32),
    )(out, V, cnt, w, b, batch)


def _final_call(POOL, cnt, lin_W, lin_b):
    D = POOL.shape[1]
    NCLS = lin_W.shape[1]

    def body(P_ref, cnt_ref, W_ref, b_ref, o_ref):
        pooled = P_ref[...] / jnp.maximum(cnt_ref[...], 1.0)[:, None]
        o_ref[...] = jnp.dot(pooled, W_ref[...], preferred_element_type=jnp.float32, precision=lax.Precision.HIGHEST) + b_ref[...][None, :]

    return pl.pallas_call(
        body,
        in_specs=[
            pl.BlockSpec((NG, D), lambda: (0, 0)),
            pl.BlockSpec((NG,), lambda: (0,)),
            pl.BlockSpec((D, NCLS), lambda: (0, 0)),
            pl.BlockSpec((NCLS,), lambda: (0,)),
        ],
        out_specs=pl.BlockSpec((NG, NCLS), lambda: (0, 0)),
        out_shape=jax.ShapeDtypeStruct((NG, NCLS), jnp.float32),
    )(POOL, cnt, lin_W, lin_b)


def kernel(x, edge_index, batch, W1, b1, gn1_weight, gn1_bias, gn1_mean_scale,
           W2, b2, gn2_weight, gn2_bias, gn2_mean_scale, lin_W, lin_b):
    N, DIN = x.shape
    D = W1.shape[1]
    E = edge_index.shape[1]
    C = 1000

    Gd = E // (NW * K)
    NPd = ((N + (16 * NS) - 1) // (16 * NS)) * (16 * NS)
    dst3d = edge_index[1].reshape(NW, Gd, K)

    KA = 128
    Ga = -(-E // (NW * KA))
    Ga = -(-Ga // 4) * 4
    Ea = NW * KA * Ga
    pad = Ea - E
    NPa = -(-N // (NS * 64)) * (NS * 64)
    src_p = jnp.concatenate(
        [edge_index[0], jnp.arange(pad, dtype=jnp.int32) % N])
    dst_p = jnp.concatenate(
        [edge_index[1], N + (jnp.arange(pad, dtype=jnp.int32) % (NPa - N))])
    src3a = src_p.reshape(NW, Ga, KA)
    dst3a = dst_p.reshape(NW, Ga, KA)

    degp = _deg_call(dst3d, NPd, Gd)
    deg0, deg1 = degp[0, :N, None], degp[1, :N, None]

    batch2 = batch[:, None]
    xws1, dis = _xw_scale_call(x, W1, deg0, deg1, C)

    aggp1 = _edge_aggregate_call(xws1, src3a, dst3a, NPa, Ga, KA, D)
    h1, S1, cnt = _post_agg_call(aggp1[0], aggp1[1], xws1, dis, b1, batch2, C)
    out1, V1 = _center_call(h1, S1, cnt, batch2, gn1_mean_scale, C)
    xws2 = _norm_relu_xw_call(out1, V1, cnt, gn1_weight, gn1_bias, dis, W2, batch2, C)

    aggp2 = _edge_aggregate_call(xws2, src3a, dst3a, NPa, Ga, KA, D)
    h2, S2, cnt2 = _post_agg_call(aggp2[0], aggp2[1], xws2, dis, b2, batch2, C)
    out2, V2 = _center_call(h2, S2, cnt2, batch2, gn2_mean_scale, C)
    POOL = _norm_relu_pool_call(out2, V2, cnt2, gn2_weight, gn2_bias, batch2, C)

    return _final_call(POOL, cnt2, lin_W, lin_b)


# speedup vs baseline: 2.7041x; 1.0588x over previous
import functools

import jax
import jax.numpy as jnp
from jax import lax
from jax.experimental import pallas as pl
from jax.experimental.pallas import tpu as pltpu
from jax.experimental.pallas import tpu_sc as plsc

EPS = 1e-5
NG = 64
NC = 2
NS = 16
NW = NC * NS
K = 80


def _deg_call(dst3, NP, G):
    mesh = plsc.VectorSubcoreMesh(core_axis_name="c", subcore_axis_name="s")
    rpt = NP // NS

    @functools.partial(
        pl.kernel,
        out_type=jax.ShapeDtypeStruct((NC, NP), jnp.float32),
        mesh=mesh,
        scratch_types=[
            pltpu.VMEM((G, K), jnp.int32),
            pltpu.VMEM((K,), jnp.float32),
            pltpu.VMEM((rpt,), jnp.float32),
            pltpu.VMEM_SHARED((NP,), jnp.float32),
        ],
    )
    def k(dst_hbm, out_hbm, idx_v, ones_v, buf_v, acc_sh):
        c = lax.axis_index("c")
        s = lax.axis_index("s")
        w = c * NS + s

        def fill_zero(i, carry):
            buf_v[pl.ds(i * 16, 16)] = jnp.zeros((16,), jnp.float32)
            return carry
        lax.fori_loop(0, rpt // 16, fill_zero, 0)

        def fill_one(i, carry):
            ones_v[pl.ds(i * 16, 16)] = jnp.ones((16,), jnp.float32)
            return carry
        lax.fori_loop(0, K // 16, fill_one, 0)

        pltpu.sync_copy(buf_v, acc_sh.at[pl.ds(s * rpt, rpt)])
        plsc.subcore_barrier()

        pltpu.sync_copy(dst_hbm.at[w], idx_v)

        def body(g, carry):
            pltpu.sync_copy(ones_v, acc_sh.at[idx_v.at[g]], add=True)
            return carry
        lax.fori_loop(0, G, body, 0)

        plsc.subcore_barrier()
        pltpu.sync_copy(acc_sh.at[pl.ds(s * rpt, rpt)], buf_v)
        pltpu.sync_copy(buf_v, out_hbm.at[c, pl.ds(s * rpt, rpt)])

    return k(dst3)


def _edge_aggregate_call(xws, src3, dst3, NP, G, KA, D):
    mesh = plsc.VectorSubcoreMesh(core_axis_name="c", subcore_axis_name="s")
    rpt = NP // NS
    ch = 64

    @functools.partial(
        pl.kernel,
        out_type=jax.ShapeDtypeStruct((NC, NP, D), jnp.float32),
        mesh=mesh,
        scratch_types=[
            pltpu.VMEM((G // 2, KA), jnp.int32),
            pltpu.VMEM((G // 2, KA), jnp.int32),
            pltpu.VMEM((KA, D), jnp.float32),
            pltpu.VMEM((KA, D), jnp.float32),
            pltpu.VMEM_SHARED((NP, D), jnp.float32),
            pltpu.SemaphoreType.DMA,
            pltpu.SemaphoreType.DMA,
        ],
    )
    def k(xws_hbm, src_hbm, dst_hbm, out_hbm, src_v, dst_v, r0, r1,
          acc_sh, gsem, ssem):
        c = lax.axis_index("c")
        s = lax.axis_index("s")
        w = c * NS + s
        rows = (r0, r1)
        G2 = G // 2

        def zrow(i, carry):
            for j in range(D // 16):
                r0[i, pl.ds(j * 16, 16)] = jnp.zeros((16,), jnp.float32)
            return carry
        lax.fori_loop(0, ch, zrow, 0)

        for t in range(rpt // ch):
            pltpu.sync_copy(r0.at[pl.ds(0, ch)], acc_sh.at[pl.ds(s * rpt + t * ch, ch)])
        plsc.subcore_barrier()

        for h in range(2):
            pltpu.sync_copy(src_hbm.at[w, pl.ds(h * G2, G2)], src_v)
            pltpu.sync_copy(dst_hbm.at[w, pl.ds(h * G2, G2)], dst_v)

            pltpu.async_copy(xws_hbm.at[src_v.at[0]], rows[0], gsem)

            def pair(p, carry):
                for b in range(2):
                    g = 2 * p + b
                    pltpu.make_async_copy(xws_hbm.at[src_v.at[g]], rows[b], gsem).wait()
                    pltpu.async_copy(rows[b], acc_sh.at[dst_v.at[g]], ssem, add=True)

                    @pl.when(g + 1 < G2)
                    def _():
                        @pl.when(g >= 1)
                        def _():
                            pltpu.make_async_copy(
                                rows[1 - b], acc_sh.at[dst_v.at[g - 1]], ssem
                            ).wait()
                        pltpu.async_copy(xws_hbm.at[src_v.at[g + 1]], rows[1 - b], gsem)
                return carry
            lax.fori_loop(0, G2 // 2, pair, 0)

            pltpu.make_async_copy(rows[G2 % 2], acc_sh.at[dst_v.at[G2 - 2]], ssem).wait()
            pltpu.make_async_copy(rows[1 - G2 % 2], acc_sh.at[dst_v.at[G2 - 1]], ssem).wait()

        plsc.subcore_barrier()
        for t in range(rpt // ch):
            pltpu.sync_copy(acc_sh.at[pl.ds(s * rpt + t * ch, ch)], r0.at[pl.ds(0, ch)])
            pltpu.sync_copy(r0.at[pl.ds(0, ch)], out_hbm.at[c, pl.ds(s * rpt + t * ch, ch)])

    return k(xws, src3, dst3)


def _xw_scale_call(x, W, deg0, deg1, C):
    N, DIN = x.shape
    D = W.shape[1]

    def body(x_ref, w_ref, d0_ref, d1_ref, xws_ref, dis_ref):
        deg = d0_ref[...] + d1_ref[...] + 1.0
        dis = lax.rsqrt(deg)
        xw = jnp.dot(x_ref[...], w_ref[...], preferred_element_type=jnp.float32, precision=lax.Precision.HIGHEST)
        xws_ref[...] = xw * dis
        dis_ref[...] = dis

    return pl.pallas_call(
        body,
        grid=(N // C,),
        in_specs=[
            pl.BlockSpec((C, DIN), lambda i: (i, 0)),
            pl.BlockSpec((DIN, D), lambda i: (0, 0)),
            pl.BlockSpec((C, 1), lambda i: (i, 0)),
            pl.BlockSpec((C, 1), lambda i: (i, 0)),
        ],
        out_specs=[
            pl.BlockSpec((C, D), lambda i: (i, 0)),
            pl.BlockSpec((C, 1), lambda i: (i, 0)),
        ],
        out_shape=[
            jax.ShapeDtypeStruct((N, D), jnp.float32),
            jax.ShapeDtypeStruct((N, 1), jnp.float32),
        ],
    )(x, W, deg0, deg1)


def _post_agg_call(a0, a1, xws, dis, b, batch, C):
    N, D = xws.shape

    def body(a0_ref, a1_ref, xws_ref, dis_ref, b_ref, bat_ref, h_ref, S_ref, Q_ref, cnt_ref):
        i = pl.program_id(0)
        h = dis_ref[...] * (a0_ref[...] + a1_ref[...] + xws_ref[...]) + b_ref[...][None, :]
        h_ref[...] = h
        oh = (lax.broadcasted_iota(jnp.int32, (NG, C), 0) == bat_ref[...][:, 0][None, :]).astype(jnp.float32)

        @pl.when(i == 0)
        def _():
            S_ref[...] = jnp.zeros_like(S_ref)
            Q_ref[...] = jnp.zeros_like(Q_ref)
            cnt_ref[...] = jnp.zeros_like(cnt_ref)

        S_ref[...] += jnp.dot(oh, h, preferred_element_type=jnp.float32, precision=lax.Precision.HIGHEST)
        Q_ref[...] += jnp.dot(oh, h * h, preferred_element_type=jnp.float32, precision=lax.Precision.HIGHEST)
        cnt_ref[...] += jnp.sum(oh, axis=1)

    return pl.pallas_call(
        body,
        grid=(N // C,),
        in_specs=[
            pl.BlockSpec((C, D), lambda i: (i, 0)),
            pl.BlockSpec((C, D), lambda i: (i, 0)),
            pl.BlockSpec((C, D), lambda i: (i, 0)),
            pl.BlockSpec((C, 1), lambda i: (i, 0)),
            pl.BlockSpec((D,), lambda i: (0,)),
            pl.BlockSpec((C, 1), lambda i: (i, 0)),
        ],
        out_specs=[
            pl.BlockSpec((C, D), lambda i: (i, 0)),
            pl.BlockSpec((NG, D), lambda i: (0, 0)),
            pl.BlockSpec((NG, D), lambda i: (0, 0)),
            pl.BlockSpec((NG,), lambda i: (0,)),
        ],
        out_shape=[
            jax.ShapeDtypeStruct((N, D), jnp.float32),
            jax.ShapeDtypeStruct((NG, D), jnp.float32),
            jax.ShapeDtypeStruct((NG, D), jnp.float32),
            jax.ShapeDtypeStruct((NG,), jnp.float32),
        ],
    )(a0, a1, xws, dis, b, batch)


def _norm_relu_xw_call(h, S, Q, cnt, batch, ms, w, b, dis, W2, C):
    N, D = h.shape

    def body(h_ref, S_ref, Q_ref, cnt_ref, bat_ref, ms_ref, w_ref, b_ref, dis_ref, W2_ref, xws_ref):
        cinv = 1.0 / jnp.maximum(cnt_ref[...], 1.0)[:, None]
        mean = S_ref[...] * cinv
        msv = ms_ref[...][None, :]
        var = Q_ref[...] * cinv - mean * mean * (msv * (2.0 - msv))
        std = jnp.sqrt(jnp.maximum(var, 0.0) + EPS)
        ohT = (lax.broadcasted_iota(jnp.int32, (C, NG), 1) == bat_ref[...][:, 0][:, None]).astype(jnp.float32)
        mb = jnp.dot(ohT, mean, preferred_element_type=jnp.float32, precision=lax.Precision.HIGHEST)
        stdb = jnp.dot(ohT, std, preferred_element_type=jnp.float32, precision=lax.Precision.HIGHEST)
        out = h_ref[...] - mb * msv
        hn = w_ref[...][None, :] * out / stdb + b_ref[...][None, :]
        hn = jnp.maximum(hn, 0.0)
        xw = jnp.dot(hn, W2_ref[...], preferred_element_type=jnp.float32, precision=lax.Precision.HIGHEST)
        xws_ref[...] = xw * dis_ref[...]

    return pl.pallas_call(
        body,
        grid=(N // C,),
        in_specs=[
            pl.BlockSpec((C, D), lambda i: (i, 0)),
            pl.BlockSpec((NG, D), lambda i: (0, 0)),
            pl.BlockSpec((NG, D), lambda i: (0, 0)),
            pl.BlockSpec((NG,), lambda i: (0,)),
            pl.BlockSpec((C, 1), lambda i: (i, 0)),
            pl.BlockSpec((D,), lambda i: (0,)),
            pl.BlockSpec((D,), lambda i: (0,)),
            pl.BlockSpec((D,), lambda i: (0,)),
            pl.BlockSpec((C, 1), lambda i: (i, 0)),
            pl.BlockSpec((D, D), lambda i: (0, 0)),
        ],
        out_specs=pl.BlockSpec((C, D), lambda i: (i, 0)),
        out_shape=jax.ShapeDtypeStruct((N, D), jnp.float32),
    )(h, S, Q, cnt, batch, ms, w, b, dis, W2)


def _norm_relu_pool_call(h, S, Q, cnt, batch, ms, w, b, C):
    N, D = h.shape

    def body(h_ref, S_ref, Q_ref, cnt_ref, bat_ref, ms_ref, w_ref, b_ref, P_ref):
        i = pl.program_id(0)
        cinv = 1.0 / jnp.maximum(cnt_ref[...], 1.0)[:, None]
        mean = S_ref[...] * cinv
        msv = ms_ref[...][None, :]
        var = Q_ref[...] * cinv - mean * mean * (msv * (2.0 - msv))
        std = jnp.sqrt(jnp.maximum(var, 0.0) + EPS)
        bat = bat_ref[...][:, 0]
        ohT = (lax.broadcasted_iota(jnp.int32, (C, NG), 1) == bat[:, None]).astype(jnp.float32)
        mb = jnp.dot(ohT, mean, preferred_element_type=jnp.float32, precision=lax.Precision.HIGHEST)
        stdb = jnp.dot(ohT, std, preferred_element_type=jnp.float32, precision=lax.Precision.HIGHEST)
        out = h_ref[...] - mb * msv
        hn = w_ref[...][None, :] * out / stdb + b_ref[...][None, :]
        hn = jnp.maximum(hn, 0.0)
        oh = (lax.broadcasted_iota(jnp.int32, (NG, C), 0) == bat[None, :]).astype(jnp.float32)

        @pl.when(i == 0)
        def _():
            P_ref[...] = jnp.zeros_like(P_ref)

        P_ref[...] += jnp.dot(oh, hn, preferred_element_type=jnp.float32, precision=lax.Precision.HIGHEST)

    return pl.pallas_call(
        body,
        grid=(N // C,),
        in_specs=[
            pl.BlockSpec((C, D), lambda i: (i, 0)),
            pl.BlockSpec((NG, D), lambda i: (0, 0)),
            pl.BlockSpec((NG, D), lambda i: (0, 0)),
            pl.BlockSpec((NG,), lambda i: (0,)),
            pl.BlockSpec((C, 1), lambda i: (i, 0)),
            pl.BlockSpec((D,), lambda i: (0,)),
            pl.BlockSpec((D,), lambda i: (0,)),
            pl.BlockSpec((D,), lambda i: (0,)),
        ],
        out_specs=pl.BlockSpec((NG, D), lambda i: (0, 0)),
        out_shape=jax.ShapeDtypeStruct((NG, D), jnp.float32),
    )(h, S, Q, cnt, batch, ms, w, b)


def _final_call(POOL, cnt, lin_W, lin_b):
    D = POOL.shape[1]
    NCLS = lin_W.shape[1]

    def body(P_ref, cnt_ref, W_ref, b_ref, o_ref):
        pooled = P_ref[...] / jnp.maximum(cnt_ref[...], 1.0)[:, None]
        o_ref[...] = jnp.dot(pooled, W_ref[...], preferred_element_type=jnp.float32, precision=lax.Precision.HIGHEST) + b_ref[...][None, :]

    return pl.pallas_call(
        body,
        in_specs=[
            pl.BlockSpec((NG, D), lambda: (0, 0)),
            pl.BlockSpec((NG,), lambda: (0,)),
            pl.BlockSpec((D, NCLS), lambda: (0, 0)),
            pl.BlockSpec((NCLS,), lambda: (0,)),
        ],
        out_specs=pl.BlockSpec((NG, NCLS), lambda: (0, 0)),
        out_shape=jax.ShapeDtypeStruct((NG, NCLS), jnp.float32),
    )(POOL, cnt, lin_W, lin_b)


def kernel(x, edge_index, batch, W1, b1, gn1_weight, gn1_bias, gn1_mean_scale,
           W2, b2, gn2_weight, gn2_bias, gn2_mean_scale, lin_W, lin_b):
    N, DIN = x.shape
    D = W1.shape[1]
    E = edge_index.shape[1]
    C = 1000

    Gd = E // (NW * K)
    NPd = ((N + (16 * NS) - 1) // (16 * NS)) * (16 * NS)
    dst3d = edge_index[1].reshape(NW, Gd, K)

    KA = 128
    Ga = -(-E // (NW * KA))
    Ga = -(-Ga // 4) * 4
    Ea = NW * KA * Ga
    pad = Ea - E
    NPa = -(-N // (NS * 64)) * (NS * 64)
    src_p = jnp.concatenate(
        [edge_index[0], jnp.arange(pad, dtype=jnp.int32) % N])
    dst_p = jnp.concatenate(
        [edge_index[1], N + (jnp.arange(pad, dtype=jnp.int32) % (NPa - N))])
    src3a = src_p.reshape(NW, Ga, KA)
    dst3a = dst_p.reshape(NW, Ga, KA)

    degp = _deg_call(dst3d, NPd, Gd)
    deg0, deg1 = degp[0, :N, None], degp[1, :N, None]

    batch2 = batch[:, None]
    xws1, dis = _xw_scale_call(x, W1, deg0, deg1, C)

    aggp1 = _edge_aggregate_call(xws1, src3a, dst3a, NPa, Ga, KA, D)
    h1, S1, Q1, cnt = _post_agg_call(aggp1[0], aggp1[1], xws1, dis, b1, batch2, C)
    xws2 = _norm_relu_xw_call(h1, S1, Q1, cnt, batch2, gn1_mean_scale,
                              gn1_weight, gn1_bias, dis, W2, C)

    aggp2 = _edge_aggregate_call(xws2, src3a, dst3a, NPa, Ga, KA, D)
    h2, S2, Q2, cnt2 = _post_agg_call(aggp2[0], aggp2[1], xws2, dis, b2, batch2, C)
    POOL = _norm_relu_pool_call(h2, S2, Q2, cnt2, batch2, gn2_mean_scale,
                                gn2_weight, gn2_bias, C)

    return _final_call(POOL, cnt2, lin_W, lin_b)


# speedup vs baseline: 2.7307x; 1.0098x over previous
import functools

import jax
import jax.numpy as jnp
from jax import lax
from jax.experimental import pallas as pl
from jax.experimental.pallas import tpu as pltpu
from jax.experimental.pallas import tpu_sc as plsc

EPS = 1e-5
NG = 64
NC = 2
NS = 16
NW = NC * NS
K = 80


def _deg_call(dst3, NP, G):
    mesh = plsc.VectorSubcoreMesh(core_axis_name="c", subcore_axis_name="s")
    rpt = NP // NS

    @functools.partial(
        pl.kernel,
        out_type=jax.ShapeDtypeStruct((NC, NP), jnp.float32),
        mesh=mesh,
        scratch_types=[
            pltpu.VMEM((G, K), jnp.int32),
            pltpu.VMEM((K,), jnp.float32),
            pltpu.VMEM((rpt,), jnp.float32),
            pltpu.VMEM_SHARED((NP,), jnp.float32),
        ],
    )
    def k(dst_hbm, out_hbm, idx_v, ones_v, buf_v, acc_sh):
        c = lax.axis_index("c")
        s = lax.axis_index("s")
        w = c * NS + s

        def fill_zero(i, carry):
            buf_v[pl.ds(i * 16, 16)] = jnp.zeros((16,), jnp.float32)
            return carry
        lax.fori_loop(0, rpt // 16, fill_zero, 0)

        def fill_one(i, carry):
            ones_v[pl.ds(i * 16, 16)] = jnp.ones((16,), jnp.float32)
            return carry
        lax.fori_loop(0, K // 16, fill_one, 0)

        pltpu.sync_copy(buf_v, acc_sh.at[pl.ds(s * rpt, rpt)])
        plsc.subcore_barrier()

        pltpu.sync_copy(dst_hbm.at[w], idx_v)

        def body(g, carry):
            pltpu.sync_copy(ones_v, acc_sh.at[idx_v.at[g]], add=True)
            return carry
        lax.fori_loop(0, G, body, 0)

        plsc.subcore_barrier()
        pltpu.sync_copy(acc_sh.at[pl.ds(s * rpt, rpt)], buf_v)
        pltpu.sync_copy(buf_v, out_hbm.at[c, pl.ds(s * rpt, rpt)])

    return k(dst3)


def _edge_aggregate_call(xws, src3, dst3, NP, G, KA, D):
    mesh = plsc.VectorSubcoreMesh(core_axis_name="c", subcore_axis_name="s")
    rpt = NP // NS
    ch = 64

    @functools.partial(
        pl.kernel,
        out_type=jax.ShapeDtypeStruct((NC, NP, D), jnp.float32),
        mesh=mesh,
        scratch_types=[
            pltpu.VMEM((G // 2, KA), jnp.int32),
            pltpu.VMEM((G // 2, KA), jnp.int32),
            pltpu.VMEM((KA, D), jnp.float32),
            pltpu.VMEM((KA, D), jnp.float32),
            pltpu.VMEM_SHARED((NP, D), jnp.float32),
            pltpu.SemaphoreType.DMA,
            pltpu.SemaphoreType.DMA,
        ],
    )
    def k(xws_hbm, src_hbm, dst_hbm, out_hbm, src_v, dst_v, r0, r1,
          acc_sh, gsem, ssem):
        c = lax.axis_index("c")
        s = lax.axis_index("s")
        w = c * NS + s
        rows = (r0, r1)
        G2 = G // 2

        def zrow(i, carry):
            for j in range(D // 16):
                r0[i, pl.ds(j * 16, 16)] = jnp.zeros((16,), jnp.float32)
            return carry
        lax.fori_loop(0, ch, zrow, 0)

        for t in range(rpt // ch):
            pltpu.sync_copy(r0.at[pl.ds(0, ch)], acc_sh.at[pl.ds(s * rpt + t * ch, ch)])
        plsc.subcore_barrier()

        for h in range(2):
            pltpu.sync_copy(src_hbm.at[w, pl.ds(h * G2, G2)], src_v)
            pltpu.sync_copy(dst_hbm.at[w, pl.ds(h * G2, G2)], dst_v)

            pltpu.async_copy(xws_hbm.at[src_v.at[0]], rows[0], gsem)

            def pair(p, carry):
                for b in range(2):
                    g = 2 * p + b
                    pltpu.make_async_copy(xws_hbm.at[src_v.at[g]], rows[b], gsem).wait()
                    pltpu.async_copy(rows[b], acc_sh.at[dst_v.at[g]], ssem, add=True)

                    @pl.when(g + 1 < G2)
                    def _():
                        @pl.when(g >= 1)
                        def _():
                            pltpu.make_async_copy(
                                rows[1 - b], acc_sh.at[dst_v.at[g - 1]], ssem
                            ).wait()
                        pltpu.async_copy(xws_hbm.at[src_v.at[g + 1]], rows[1 - b], gsem)
                return carry
            lax.fori_loop(0, G2 // 2, pair, 0)

            pltpu.make_async_copy(rows[G2 % 2], acc_sh.at[dst_v.at[G2 - 2]], ssem).wait()
            pltpu.make_async_copy(rows[1 - G2 % 2], acc_sh.at[dst_v.at[G2 - 1]], ssem).wait()

        plsc.subcore_barrier()
        for t in range(rpt // ch):
            pltpu.sync_copy(acc_sh.at[pl.ds(s * rpt + t * ch, ch)], r0.at[pl.ds(0, ch)])
            pltpu.sync_copy(r0.at[pl.ds(0, ch)], out_hbm.at[c, pl.ds(s * rpt + t * ch, ch)])

    return k(xws, src3, dst3)


def _xw_scale_call(x, W, deg0, deg1, C):
    N, DIN = x.shape
    D = W.shape[1]

    def body(x_ref, w_ref, d0_ref, d1_ref, xws_ref, dis_ref):
        deg = d0_ref[...] + d1_ref[...] + 1.0
        dis = lax.rsqrt(deg)
        xw = jnp.dot(x_ref[...], w_ref[...], preferred_element_type=jnp.float32, precision=lax.Precision.HIGHEST)
        xws_ref[...] = xw * dis
        dis_ref[...] = dis

    return pl.pallas_call(
        body,
        grid=(N // C,),
        in_specs=[
            pl.BlockSpec((C, DIN), lambda i: (i, 0)),
            pl.BlockSpec((DIN, D), lambda i: (0, 0)),
            pl.BlockSpec((C, 1), lambda i: (i, 0)),
            pl.BlockSpec((C, 1), lambda i: (i, 0)),
        ],
        out_specs=[
            pl.BlockSpec((C, D), lambda i: (i, 0)),
            pl.BlockSpec((C, 1), lambda i: (i, 0)),
        ],
        out_shape=[
            jax.ShapeDtypeStruct((N, D), jnp.float32),
            jax.ShapeDtypeStruct((N, 1), jnp.float32),
        ],
    )(x, W, deg0, deg1)


def _graphnorm_stats(S_scr, Q_scr, cnt_scr, msv):
    cinv = 1.0 / jnp.maximum(cnt_scr[...], 1.0)
    mean = S_scr[...] * cinv
    var = Q_scr[...] * cinv - mean * mean * (msv * (2.0 - msv))
    std = jnp.sqrt(jnp.maximum(var, 0.0) + EPS)
    return mean, std


def _layer_mid_call(a0, a1, xws, dis, b, batch, ms, w, gb, W2, C):
    N, D = xws.shape
    NB = N // C

    def body(a0_ref, a1_ref, xws_ref, dis_ref, b_ref, bat_ref, ms_ref, w_ref,
             gb_ref, W2_ref, o_ref, h_scr, S_scr, Q_scr, cnt_scr):
        p = pl.program_id(0)
        i = pl.program_id(1)
        bat = bat_ref[...][:, 0]

        @pl.when(p == 0)
        def _():
            h = dis_ref[...] * (a0_ref[...] + a1_ref[...] + xws_ref[...]) + b_ref[...][None, :]
            h_scr[pl.ds(i * C, C), :] = h
            oh = (lax.broadcasted_iota(jnp.int32, (NG, C), 0) == bat[None, :]).astype(jnp.float32)

            @pl.when(i == 0)
            def _():
                S_scr[...] = jnp.zeros_like(S_scr)
                Q_scr[...] = jnp.zeros_like(Q_scr)
                cnt_scr[...] = jnp.zeros_like(cnt_scr)

            S_scr[...] += jnp.dot(oh, h, preferred_element_type=jnp.float32, precision=lax.Precision.HIGHEST)
            Q_scr[...] += jnp.dot(oh, h * h, preferred_element_type=jnp.float32, precision=lax.Precision.HIGHEST)
            cnt_scr[...] += jnp.sum(oh, axis=1)[:, None]

        @pl.when(p == 1)
        def _():
            msv = ms_ref[...][None, :]
            mean, std = _graphnorm_stats(S_scr, Q_scr, cnt_scr, msv)
            ohT = (lax.broadcasted_iota(jnp.int32, (C, NG), 1) == bat[:, None]).astype(jnp.float32)
            mb = jnp.dot(ohT, mean, preferred_element_type=jnp.float32, precision=lax.Precision.HIGHEST)
            stdb = jnp.dot(ohT, std, preferred_element_type=jnp.float32, precision=lax.Precision.HIGHEST)
            out = h_scr[pl.ds(i * C, C), :] - mb * msv
            hn = jnp.maximum(w_ref[...][None, :] * out / stdb + gb_ref[...][None, :], 0.0)
            xw = jnp.dot(hn, W2_ref[...], preferred_element_type=jnp.float32, precision=lax.Precision.HIGHEST)
            o_ref[...] = xw * dis_ref[...]

    return pl.pallas_call(
        body,
        grid=(2, NB),
        in_specs=[
            pl.BlockSpec((C, D), lambda p, i: (i * (1 - p), 0)),
            pl.BlockSpec((C, D), lambda p, i: (i * (1 - p), 0)),
            pl.BlockSpec((C, D), lambda p, i: (i * (1 - p), 0)),
            pl.BlockSpec((C, 1), lambda p, i: (i, 0)),
            pl.BlockSpec((D,), lambda p, i: (0,)),
            pl.BlockSpec((C, 1), lambda p, i: (i, 0)),
            pl.BlockSpec((D,), lambda p, i: (0,)),
            pl.BlockSpec((D,), lambda p, i: (0,)),
            pl.BlockSpec((D,), lambda p, i: (0,)),
            pl.BlockSpec((D, D), lambda p, i: (0, 0)),
        ],
        out_specs=pl.BlockSpec((C, D), lambda p, i: (i * p, 0)),
        out_shape=jax.ShapeDtypeStruct((N, D), jnp.float32),
        scratch_shapes=[
            pltpu.VMEM((N, D), jnp.float32),
            pltpu.VMEM((NG, D), jnp.float32),
            pltpu.VMEM((NG, D), jnp.float32),
            pltpu.VMEM((NG, 1), jnp.float32),
        ],
    )(a0, a1, xws, dis, b, batch, ms, w, gb, W2)


def _layer_out_call(a0, a1, xws, dis, b, batch, ms, w, gb, lin_W, lin_b, C):
    N, D = xws.shape
    NB = N // C
    NCLS = lin_W.shape[1]

    def body(a0_ref, a1_ref, xws_ref, dis_ref, b_ref, bat_ref, ms_ref, w_ref,
             gb_ref, lW_ref, lb_ref, o_ref, h_scr, S_scr, Q_scr, cnt_scr, P_scr):
        p = pl.program_id(0)
        i = pl.program_id(1)
        bat = bat_ref[...][:, 0]

        @pl.when(p == 0)
        def _():
            h = dis_ref[...] * (a0_ref[...] + a1_ref[...] + xws_ref[...]) + b_ref[...][None, :]
            h_scr[pl.ds(i * C, C), :] = h
            oh = (lax.broadcasted_iota(jnp.int32, (NG, C), 0) == bat[None, :]).astype(jnp.float32)

            @pl.when(i == 0)
            def _():
                S_scr[...] = jnp.zeros_like(S_scr)
                Q_scr[...] = jnp.zeros_like(Q_scr)
                cnt_scr[...] = jnp.zeros_like(cnt_scr)

            S_scr[...] += jnp.dot(oh, h, preferred_element_type=jnp.float32, precision=lax.Precision.HIGHEST)
            Q_scr[...] += jnp.dot(oh, h * h, preferred_element_type=jnp.float32, precision=lax.Precision.HIGHEST)
            cnt_scr[...] += jnp.sum(oh, axis=1)[:, None]

        @pl.when(p == 1)
        def _():
            msv = ms_ref[...][None, :]
            mean, std = _graphnorm_stats(S_scr, Q_scr, cnt_scr, msv)
            ohT = (lax.broadcasted_iota(jnp.int32, (C, NG), 1) == bat[:, None]).astype(jnp.float32)
            mb = jnp.dot(ohT, mean, preferred_element_type=jnp.float32, precision=lax.Precision.HIGHEST)
            stdb = jnp.dot(ohT, std, preferred_element_type=jnp.float32, precision=lax.Precision.HIGHEST)
            out = h_scr[pl.ds(i * C, C), :] - mb * msv
            hn = jnp.maximum(w_ref[...][None, :] * out / stdb + gb_ref[...][None, :], 0.0)
            oh = (lax.broadcasted_iota(jnp.int32, (NG, C), 0) == bat[None, :]).astype(jnp.float32)

            @pl.when(i == 0)
            def _():
                P_scr[...] = jnp.zeros_like(P_scr)

            P_scr[...] += jnp.dot(oh, hn, preferred_element_type=jnp.float32, precision=lax.Precision.HIGHEST)

            @pl.when(i == NB - 1)
            def _():
                pooled = P_scr[...] / jnp.maximum(cnt_scr[...], 1.0)
                o_ref[...] = jnp.dot(pooled, lW_ref[...], preferred_element_type=jnp.float32, precision=lax.Precision.HIGHEST) + lb_ref[...][None, :]

    return pl.pallas_call(
        body,
        grid=(2, NB),
        in_specs=[
            pl.BlockSpec((C, D), lambda p, i: (i * (1 - p), 0)),
            pl.BlockSpec((C, D), lambda p, i: (i * (1 - p), 0)),
            pl.BlockSpec((C, D), lambda p, i: (i * (1 - p), 0)),
            pl.BlockSpec((C, 1), lambda p, i: (i * (1 - p), 0)),
            pl.BlockSpec((D,), lambda p, i: (0,)),
            pl.BlockSpec((C, 1), lambda p, i: (i, 0)),
            pl.BlockSpec((D,), lambda p, i: (0,)),
            pl.BlockSpec((D,), lambda p, i: (0,)),
            pl.BlockSpec((D,), lambda p, i: (0,)),
            pl.BlockSpec((D, NCLS), lambda p, i: (0, 0)),
            pl.BlockSpec((NCLS,), lambda p, i: (0,)),
        ],
        out_specs=pl.BlockSpec((NG, NCLS), lambda p, i: (0, 0)),
        out_shape=jax.ShapeDtypeStruct((NG, NCLS), jnp.float32),
        scratch_shapes=[
            pltpu.VMEM((N, D), jnp.float32),
            pltpu.VMEM((NG, D), jnp.float32),
            pltpu.VMEM((NG, D), jnp.float32),
            pltpu.VMEM((NG, 1), jnp.float32),
            pltpu.VMEM((NG, D), jnp.float32),
        ],
    )(a0, a1, xws, dis, b, batch, ms, w, gb, lin_W, lin_b)


def kernel(x, edge_index, batch, W1, b1, gn1_weight, gn1_bias, gn1_mean_scale,
           W2, b2, gn2_weight, gn2_bias, gn2_mean_scale, lin_W, lin_b):
    N, DIN = x.shape
    D = W1.shape[1]
    E = edge_index.shape[1]
    C = 1000

    Gd = E // (NW * K)
    NPd = ((N + (16 * NS) - 1) // (16 * NS)) * (16 * NS)
    dst3d = edge_index[1].reshape(NW, Gd, K)

    KA = 128
    Ga = -(-E // (NW * KA))
    Ga = -(-Ga // 4) * 4
    Ea = NW * KA * Ga
    pad = Ea - E
    NPa = -(-N // (NS * 64)) * (NS * 64)
    src_p = jnp.concatenate(
        [edge_index[0], jnp.arange(pad, dtype=jnp.int32) % N])
    dst_p = jnp.concatenate(
        [edge_index[1], N + (jnp.arange(pad, dtype=jnp.int32) % (NPa - N))])
    src3a = src_p.reshape(NW, Ga, KA)
    dst3a = dst_p.reshape(NW, Ga, KA)

    degp = _deg_call(dst3d, NPd, Gd)
    deg0, deg1 = degp[0, :N, None], degp[1, :N, None]

    batch2 = batch[:, None]
    xws1, dis = _xw_scale_call(x, W1, deg0, deg1, C)

    aggp1 = _edge_aggregate_call(xws1, src3a, dst3a, NPa, Ga, KA, D)
    xws2 = _layer_mid_call(aggp1[0], aggp1[1], xws1, dis, b1, batch2,
                           gn1_mean_scale, gn1_weight, gn1_bias, W2, C)

    aggp2 = _edge_aggregate_call(xws2, src3a, dst3a, NPa, Ga, KA, D)
    return _layer_out_call(aggp2[0], aggp2[1], xws2, dis, b2, batch2,
                           gn2_mean_scale, gn2_weight, gn2_bias, lin_W, lin_b, C)


# speedup vs baseline: 3.0027x; 1.0996x over previous
import functools

import jax
import jax.numpy as jnp
from jax import lax
from jax.experimental import pallas as pl
from jax.experimental.pallas import tpu as pltpu
from jax.experimental.pallas import tpu_sc as plsc

EPS = 1e-5
NG = 64
NC = 2
NS = 16
NW = NC * NS
K = 80


def _deg_call(dst3, NP, G):
    mesh = plsc.VectorSubcoreMesh(core_axis_name="c", subcore_axis_name="s")
    rpt = NP // NS

    @functools.partial(
        pl.kernel,
        out_type=jax.ShapeDtypeStruct((NC, NP), jnp.float32),
        mesh=mesh,
        scratch_types=[
            pltpu.VMEM((G, K), jnp.int32),
            pltpu.VMEM((K,), jnp.float32),
            pltpu.VMEM((rpt,), jnp.float32),
            pltpu.VMEM_SHARED((NP,), jnp.float32),
        ],
    )
    def k(dst_hbm, out_hbm, idx_v, ones_v, buf_v, acc_sh):
        c = lax.axis_index("c")
        s = lax.axis_index("s")
        w = c * NS + s

        def fill_zero(i, carry):
            buf_v[pl.ds(i * 16, 16)] = jnp.zeros((16,), jnp.float32)
            return carry
        lax.fori_loop(0, rpt // 16, fill_zero, 0)

        def fill_one(i, carry):
            ones_v[pl.ds(i * 16, 16)] = jnp.ones((16,), jnp.float32)
            return carry
        lax.fori_loop(0, K // 16, fill_one, 0)

        pltpu.sync_copy(buf_v, acc_sh.at[pl.ds(s * rpt, rpt)])
        plsc.subcore_barrier()

        pltpu.sync_copy(dst_hbm.at[w], idx_v)

        def body(g, carry):
            pltpu.sync_copy(ones_v, acc_sh.at[idx_v.at[g]], add=True)
            return carry
        lax.fori_loop(0, G, body, 0)

        plsc.subcore_barrier()
        pltpu.sync_copy(acc_sh.at[pl.ds(s * rpt, rpt)], buf_v)
        pltpu.sync_copy(buf_v, out_hbm.at[c, pl.ds(s * rpt, rpt)])

    return k(dst3)


def _edge_aggregate_call(xws, src3, dst3, NP, G, KA, D):
    mesh = plsc.VectorSubcoreMesh(core_axis_name="c", subcore_axis_name="s")
    rpt = NP // NS
    ch = 64

    @functools.partial(
        pl.kernel,
        out_type=jax.ShapeDtypeStruct((NC, NP, D), jnp.float32),
        mesh=mesh,
        scratch_types=[
            pltpu.VMEM((G // 2, KA), jnp.int32),
            pltpu.VMEM((G // 2, KA), jnp.int32),
            pltpu.VMEM((KA, D), jnp.float32),
            pltpu.VMEM((KA, D), jnp.float32),
            pltpu.VMEM_SHARED((NP, D), jnp.float32),
            pltpu.SemaphoreType.DMA,
            pltpu.SemaphoreType.DMA,
        ],
    )
    def k(xws_hbm, src_hbm, dst_hbm, out_hbm, src_v, dst_v, r0, r1,
          acc_sh, gsem, ssem):
        c = lax.axis_index("c")
        s = lax.axis_index("s")
        w = c * NS + s
        rows = (r0, r1)
        G2 = G // 2

        def zrow(i, carry):
            for j in range(D // 16):
                r0[i, pl.ds(j * 16, 16)] = jnp.zeros((16,), jnp.float32)
            return carry
        lax.fori_loop(0, ch, zrow, 0)

        for t in range(rpt // ch):
            pltpu.sync_copy(r0.at[pl.ds(0, ch)], acc_sh.at[pl.ds(s * rpt + t * ch, ch)])
        plsc.subcore_barrier()

        for h in range(2):
            pltpu.sync_copy(src_hbm.at[w, pl.ds(h * G2, G2)], src_v)
            pltpu.sync_copy(dst_hbm.at[w, pl.ds(h * G2, G2)], dst_v)

            pltpu.async_copy(xws_hbm.at[src_v.at[0]], rows[0], gsem)

            def pair(p, carry):
                for b in range(2):
                    g = 2 * p + b
                    pltpu.make_async_copy(xws_hbm.at[src_v.at[g]], rows[b], gsem).wait()
                    pltpu.async_copy(rows[b], acc_sh.at[dst_v.at[g]], ssem, add=True)

                    @pl.when(g + 1 < G2)
                    def _():
                        @pl.when(g >= 1)
                        def _():
                            pltpu.make_async_copy(
                                rows[1 - b], acc_sh.at[dst_v.at[g - 1]], ssem
                            ).wait()
                        pltpu.async_copy(xws_hbm.at[src_v.at[g + 1]], rows[1 - b], gsem)
                return carry
            lax.fori_loop(0, G2 // 2, pair, 0)

            pltpu.make_async_copy(rows[G2 % 2], acc_sh.at[dst_v.at[G2 - 2]], ssem).wait()
            pltpu.make_async_copy(rows[1 - G2 % 2], acc_sh.at[dst_v.at[G2 - 1]], ssem).wait()

        plsc.subcore_barrier()
        for t in range(rpt // ch):
            pltpu.sync_copy(acc_sh.at[pl.ds(s * rpt + t * ch, ch)], r0.at[pl.ds(0, ch)])
            pltpu.sync_copy(r0.at[pl.ds(0, ch)], out_hbm.at[c, pl.ds(s * rpt + t * ch, ch)])

    return k(xws, src3, dst3)


def _xw_scale_call(x, W, deg0, deg1, C):
    N, DIN = x.shape
    D = W.shape[1]

    def body(x_ref, w_ref, d0_ref, d1_ref, xws_ref, dis_ref):
        deg = d0_ref[...] + d1_ref[...] + 1.0
        dis = lax.rsqrt(deg)
        xw = jnp.dot(x_ref[...], w_ref[...], preferred_element_type=jnp.float32, precision=lax.Precision.HIGHEST)
        xws_ref[...] = xw * dis
        dis_ref[...] = dis

    return pl.pallas_call(
        body,
        grid=(N // C,),
        in_specs=[
            pl.BlockSpec((C, DIN), lambda i: (i, 0)),
            pl.BlockSpec((DIN, D), lambda i: (0, 0)),
            pl.BlockSpec((C, 1), lambda i: (i, 0)),
            pl.BlockSpec((C, 1), lambda i: (i, 0)),
        ],
        out_specs=[
            pl.BlockSpec((C, D), lambda i: (i, 0)),
            pl.BlockSpec((C, 1), lambda i: (i, 0)),
        ],
        out_shape=[
            jax.ShapeDtypeStruct((N, D), jnp.float32),
            jax.ShapeDtypeStruct((N, 1), jnp.float32),
        ],
    )(x, W, deg0, deg1)


def _graphnorm_stats(S_scr, Q_scr, cnt_scr, msv):
    cinv = 1.0 / jnp.maximum(cnt_scr[...], 1.0)
    mean = S_scr[...] * cinv
    var = Q_scr[...] * cinv - mean * mean * (msv * (2.0 - msv))
    std = jnp.sqrt(jnp.maximum(var, 0.0) + EPS)
    return mean, std


def _layer_mid_call(a0, a1, xws, dis, b, batch, ms, w, gb, W2, C):
    N, D = xws.shape
    NB = N // C

    def body(a0_ref, a1_ref, xws_ref, dis_ref, b_ref, bat_ref, ms_ref, w_ref,
             gb_ref, W2_ref, o_ref, h_scr, S_scr, Q_scr, cnt_scr):
        p = pl.program_id(0)
        i = pl.program_id(1)
        bat = bat_ref[...][:, 0]

        @pl.when(p == 0)
        def _():
            h = dis_ref[...] * (a0_ref[...] + a1_ref[...] + xws_ref[...]) + b_ref[...][None, :]
            h_scr[pl.ds(i * C, C), :] = h
            oh = (lax.broadcasted_iota(jnp.int32, (NG, C), 0) == bat[None, :]).astype(jnp.float32)

            @pl.when(i == 0)
            def _():
                S_scr[...] = jnp.zeros_like(S_scr)
                Q_scr[...] = jnp.zeros_like(Q_scr)
                cnt_scr[...] = jnp.zeros_like(cnt_scr)

            S_scr[...] += jnp.dot(oh, h, preferred_element_type=jnp.float32, precision=lax.Precision.HIGHEST)
            Q_scr[...] += jnp.dot(oh, h * h, preferred_element_type=jnp.float32, precision=lax.Precision.HIGHEST)
            cnt_scr[...] += jnp.sum(oh, axis=1)[:, None]

        @pl.when(p == 1)
        def _():
            msv = ms_ref[...][None, :]
            mean, std = _graphnorm_stats(S_scr, Q_scr, cnt_scr, msv)
            ohT = (lax.broadcasted_iota(jnp.int32, (C, NG), 1) == bat[:, None]).astype(jnp.float32)
            mb = jnp.dot(ohT, mean, preferred_element_type=jnp.float32, precision=lax.Precision.HIGHEST)
            stdb = jnp.dot(ohT, std, preferred_element_type=jnp.float32, precision=lax.Precision.HIGHEST)
            out = h_scr[pl.ds(i * C, C), :] - mb * msv
            hn = jnp.maximum(w_ref[...][None, :] * out / stdb + gb_ref[...][None, :], 0.0)
            xw = jnp.dot(hn, W2_ref[...], preferred_element_type=jnp.float32, precision=lax.Precision.HIGHEST)
            o_ref[...] = xw * dis_ref[...]

    return pl.pallas_call(
        body,
        grid=(2, NB),
        in_specs=[
            pl.BlockSpec((C, D), lambda p, i: (i * (1 - p), 0)),
            pl.BlockSpec((C, D), lambda p, i: (i * (1 - p), 0)),
            pl.BlockSpec((C, D), lambda p, i: (i * (1 - p), 0)),
            pl.BlockSpec((C, 1), lambda p, i: (i, 0)),
            pl.BlockSpec((D,), lambda p, i: (0,)),
            pl.BlockSpec((C, 1), lambda p, i: (i, 0)),
            pl.BlockSpec((D,), lambda p, i: (0,)),
            pl.BlockSpec((D,), lambda p, i: (0,)),
            pl.BlockSpec((D,), lambda p, i: (0,)),
            pl.BlockSpec((D, D), lambda p, i: (0, 0)),
        ],
        out_specs=pl.BlockSpec((C, D), lambda p, i: (i * p, 0)),
        out_shape=jax.ShapeDtypeStruct((N, D), jnp.float32),
        scratch_shapes=[
            pltpu.VMEM((N, D), jnp.float32),
            pltpu.VMEM((NG, D), jnp.float32),
            pltpu.VMEM((NG, D), jnp.float32),
            pltpu.VMEM((NG, 1), jnp.float32),
        ],
    )(a0, a1, xws, dis, b, batch, ms, w, gb, W2)


def _layer_out_call(a0, a1, xws, dis, b, batch, ms, w, gb, lin_W, lin_b, C):
    N, D = xws.shape
    NB = N // C
    NCLS = lin_W.shape[1]

    def body(a0_ref, a1_ref, xws_ref, dis_ref, b_ref, bat_ref, ms_ref, w_ref,
             gb_ref, lW_ref, lb_ref, o_ref, h_scr, S_scr, Q_scr, cnt_scr, P_scr):
        p = pl.program_id(0)
        i = pl.program_id(1)
        bat = bat_ref[...][:, 0]

        @pl.when(p == 0)
        def _():
            h = dis_ref[...] * (a0_ref[...] + a1_ref[...] + xws_ref[...]) + b_ref[...][None, :]
            h_scr[pl.ds(i * C, C), :] = h
            oh = (lax.broadcasted_iota(jnp.int32, (NG, C), 0) == bat[None, :]).astype(jnp.float32)

            @pl.when(i == 0)
            def _():
                S_scr[...] = jnp.zeros_like(S_scr)
                Q_scr[...] = jnp.zeros_like(Q_scr)
                cnt_scr[...] = jnp.zeros_like(cnt_scr)

            S_scr[...] += jnp.dot(oh, h, preferred_element_type=jnp.float32, precision=lax.Precision.HIGHEST)
            Q_scr[...] += jnp.dot(oh, h * h, preferred_element_type=jnp.float32, precision=lax.Precision.HIGHEST)
            cnt_scr[...] += jnp.sum(oh, axis=1)[:, None]

        @pl.when(p == 1)
        def _():
            msv = ms_ref[...][None, :]
            mean, std = _graphnorm_stats(S_scr, Q_scr, cnt_scr, msv)
            ohT = (lax.broadcasted_iota(jnp.int32, (C, NG), 1) == bat[:, None]).astype(jnp.float32)
            mb = jnp.dot(ohT, mean, preferred_element_type=jnp.float32, precision=lax.Precision.HIGHEST)
            stdb = jnp.dot(ohT, std, preferred_element_type=jnp.float32, precision=lax.Precision.HIGHEST)
            out = h_scr[pl.ds(i * C, C), :] - mb * msv
            hn = jnp.maximum(w_ref[...][None, :] * out / stdb + gb_ref[...][None, :], 0.0)
            oh = (lax.broadcasted_iota(jnp.int32, (NG, C), 0) == bat[None, :]).astype(jnp.float32)

            @pl.when(i == 0)
            def _():
                P_scr[...] = jnp.zeros_like(P_scr)

            P_scr[...] += jnp.dot(oh, hn, preferred_element_type=jnp.float32, precision=lax.Precision.HIGHEST)

            @pl.when(i == NB - 1)
            def _():
                pooled = P_scr[...] / jnp.maximum(cnt_scr[...], 1.0)
                o_ref[...] = jnp.dot(pooled, lW_ref[...], preferred_element_type=jnp.float32, precision=lax.Precision.HIGHEST) + lb_ref[...][None, :]

    return pl.pallas_call(
        body,
        grid=(2, NB),
        in_specs=[
            pl.BlockSpec((C, D), lambda p, i: (i * (1 - p), 0)),
            pl.BlockSpec((C, D), lambda p, i: (i * (1 - p), 0)),
            pl.BlockSpec((C, D), lambda p, i: (i * (1 - p), 0)),
            pl.BlockSpec((C, 1), lambda p, i: (i * (1 - p), 0)),
            pl.BlockSpec((D,), lambda p, i: (0,)),
            pl.BlockSpec((C, 1), lambda p, i: (i, 0)),
            pl.BlockSpec((D,), lambda p, i: (0,)),
            pl.BlockSpec((D,), lambda p, i: (0,)),
            pl.BlockSpec((D,), lambda p, i: (0,)),
            pl.BlockSpec((D, NCLS), lambda p, i: (0, 0)),
            pl.BlockSpec((NCLS,), lambda p, i: (0,)),
        ],
        out_specs=pl.BlockSpec((NG, NCLS), lambda p, i: (0, 0)),
        out_shape=jax.ShapeDtypeStruct((NG, NCLS), jnp.float32),
        scratch_shapes=[
            pltpu.VMEM((N, D), jnp.float32),
            pltpu.VMEM((NG, D), jnp.float32),
            pltpu.VMEM((NG, D), jnp.float32),
            pltpu.VMEM((NG, 1), jnp.float32),
            pltpu.VMEM((NG, D), jnp.float32),
        ],
    )(a0, a1, xws, dis, b, batch, ms, w, gb, lin_W, lin_b)


def kernel(x, edge_index, batch, W1, b1, gn1_weight, gn1_bias, gn1_mean_scale,
           W2, b2, gn2_weight, gn2_bias, gn2_mean_scale, lin_W, lin_b):
    N, DIN = x.shape
    D = W1.shape[1]
    E = edge_index.shape[1]
    C = 2000

    Gd = E // (NW * K)
    NPd = ((N + (16 * NS) - 1) // (16 * NS)) * (16 * NS)
    dst3d = edge_index[1].reshape(NW, Gd, K)

    KA = 128
    Ga = -(-E // (NW * KA))
    Ga = -(-Ga // 4) * 4
    Ea = NW * KA * Ga
    pad = Ea - E
    NPa = -(-N // (NS * 64)) * (NS * 64)
    src_p = jnp.concatenate(
        [edge_index[0], jnp.arange(pad, dtype=jnp.int32) % N])
    dst_p = jnp.concatenate(
        [edge_index[1], N + (jnp.arange(pad, dtype=jnp.int32) % (NPa - N))])
    src3a = src_p.reshape(NW, Ga, KA)
    dst3a = dst_p.reshape(NW, Ga, KA)

    degp = _deg_call(dst3d, NPd, Gd)
    deg0, deg1 = degp[0, :N, None], degp[1, :N, None]

    batch2 = batch[:, None]
    xws1, dis = _xw_scale_call(x, W1, deg0, deg1, C)

    aggp1 = _edge_aggregate_call(xws1, src3a, dst3a, NPa, Ga, KA, D)
    xws2 = _layer_mid_call(aggp1[0], aggp1[1], xws1, dis, b1, batch2,
                           gn1_mean_scale, gn1_weight, gn1_bias, W2, C)

    aggp2 = _edge_aggregate_call(xws2, src3a, dst3a, NPa, Ga, KA, D)
    return _layer_out_call(aggp2[0], aggp2[1], xws2, dis, b2, batch2,
                           gn2_mean_scale, gn2_weight, gn2_bias, lin_W, lin_b, C)
